# baseline (speedup 1.0000x reference)
import functools

import jax
import jax.numpy as jnp
from jax import lax
from jax.experimental import pallas as pl
from jax.experimental.pallas import tpu as pltpu
from jax.experimental.pallas import tpu_sc as plsc

N = 50000
E = 800000
H = 4
DH = 32
H0 = 64
H1 = 128

NP = 50176
EP = 819200
NBLK = 98
RB = 512
EB = 1024
EBLK = EP // EB

SC1_HALF = 25000
SC1_ROWS = 25088
SC1_STRIPE = 1568
SC1_W = 72
SC3_CHUNK = 12512
SC3_ROWS = 12544
SC3_STRIPE = 784
SC3_W = 144
EDGES_PER_TILE = EP // 16
B = 128
NBLOCKS_TILE = EDGES_PER_TILE // B
B3 = 16
NBLOCKS3_TILE = EDGES_PER_TILE // B3


def _ln(h, g, b, eps=1e-5):
    mu = jnp.mean(h, axis=-1, keepdims=True)
    va = jnp.mean((h - mu) * (h - mu), axis=-1, keepdims=True)
    return (h - mu) / jnp.sqrt(va + eps) * g + b


def _lrelu(v):
    return jnp.maximum(v, 0.2 * v)


def _tc_face_body(xb, Wf8, bf, gf, betaf, hb):
    y = jax.nn.relu(jnp.dot(xb[...], Wf8[...],
                            preferred_element_type=jnp.float32) + bf[...])
    hb[...] = _ln(y, gf[...], betaf[...])


def _tc_edge_body(eab, We8, be, ge, betae, eepb):
    y = jax.nn.relu(jnp.dot(eab[...], We8[...],
                            preferred_element_type=jnp.float32) + be[...])
    ee = _ln(y, ge[...], betae[...])
    ones = jnp.ones((EB, 1), jnp.float32)
    zeros = jnp.zeros((EB, SC1_W - H0 - 1), jnp.float32)
    eepb[...] = jnp.concatenate([ee, ones, zeros], axis=1)


def _tc_mid_body(hb, sscb, Wfc, Al, Ar, be, ge, betae, whelb, dtabb, mx16b):
    i = pl.program_id(0)
    c0 = _ln(jax.nn.relu(be[...]), ge[...], betae[...])
    ssum = sscb[:, :H0]
    cnt = sscb[:, H0:H0 + 1]
    h2 = hb[...] + (ssum + c0) / (cnt + 1.0)
    Wh = jnp.dot(h2, Wfc[...], preferred_element_type=jnp.float32)
    el = jnp.dot(Wh, Al[...], preferred_element_type=jnp.float32)
    er = jnp.dot(Wh, Ar[...], preferred_element_type=jnp.float32)
    z12 = jnp.zeros((RB, 12), jnp.float32)
    whelb[...] = jnp.concatenate([Wh, el, z12], axis=1)
    dtabb[...] = jnp.concatenate([er, z12], axis=1)
    elz = jnp.concatenate([el, z12], axis=1)
    bm = jnp.max(elz, axis=0, keepdims=True)

    @pl.when(i == 0)
    def _():
        mx16b[...] = jnp.full((1, 16), -1e30, jnp.float32)

    mx16b[...] = jnp.maximum(mx16b[...], bm)


def _tc_post_body(ndb, whelb, dtabb, mx16b, S4, bgat, g1, beta1, Wg8, bg8,
                  nfb, z8b):
    Wh = whelb[:, :H1]
    elv = whelb[:, H1:H1 + H]
    erv = dtabb[:, :H]
    mx = mx16b[:, :H]
    sE = _lrelu(elv + erv)
    M = _lrelu(mx + erv)
    sw = jnp.exp(sE - M)
    num = ndb[:, :H1] + jnp.dot(sw, S4[...],
                                preferred_element_type=jnp.float32) * Wh
    den = ndb[:, H1:H1 + H] + sw
    out = num / (jnp.dot(den, S4[...],
                         preferred_element_type=jnp.float32) + 1e-9)
    out = jax.nn.relu(out + bgat[...])
    nf = _ln(out, g1[...], beta1[...])
    nfb[...] = nf
    z8b[...] = jnp.dot(nf, Wg8[...], preferred_element_type=jnp.float32) + bg8[...]


def _tc_zmax_body(z8b, zmaxb):
    i = pl.program_id(0)
    rows = lax.broadcasted_iota(jnp.int32, (RB, 1), 0) + i * RB
    zm = jnp.where(rows < N, z8b[:, 0:1], -1e30)
    bm = jnp.max(zm)

    @pl.when(i == 0)
    def _():
        zmaxb[...] = jnp.full((1, 8), -1e30, jnp.float32)

    zmaxb[...] = jnp.maximum(zmaxb[...], bm)


def _tc_pool_body(nfb, z8b, zmaxb, Wo, bo, go, betao, pooledb, featb,
                  Sv, S1):
    i = pl.program_id(0)

    @pl.when(i == 0)
    def _():
        Sv[...] = jnp.zeros((1, H1), jnp.float32)
        S1[...] = jnp.zeros((1, 8), jnp.float32)

    rows = lax.broadcasted_iota(jnp.int32, (RB, 1), 0) + i * RB
    ge_ = jnp.where(rows < N, jnp.exp(z8b[:, 0:1] - zmaxb[:, 0:1]), 0.0)
    Sv[...] += jnp.sum(ge_ * nfb[...], axis=0, keepdims=True)
    S1[...] += jnp.full((1, 8), jnp.sum(ge_), jnp.float32)

    @pl.when(i == NBLK - 1)
    def _():
        pooled = Sv[...] / S1[:, 0:1]
        feat = jax.nn.relu(jnp.dot(pooled, Wo[...],
                                   preferred_element_type=jnp.float32) + bo[...])
        featb[...] = _ln(feat, go[...], betao[...])
        pooledb[...] = pooled


def _row_spec(w):
    return pl.BlockSpec((1, w), lambda i: (0, 0))


_SC_MESH = plsc.VectorSubcoreMesh(core_axis_name="c", subcore_axis_name="s")
_SC_PARAMS = pltpu.CompilerParams(use_tc_tiling_on_sc=False)


def _sc1_body(eep_hbm, dst_hbm, z1_hbm, out_hbm, dstv, idxv, valv, accum):
    c = lax.axis_index("c")
    s = lax.axis_index("s")
    base = c * SC1_HALF
    pltpu.sync_copy(z1_hbm, accum.at[pl.ds(s * SC1_STRIPE, SC1_STRIPE), :])
    plsc.subcore_barrier()

    tile_off = s * EDGES_PER_TILE

    @pl.loop(0, NBLOCKS_TILE)
    def _blk(blk):
        off = tile_off + blk * B
        pltpu.sync_copy(dst_hbm.at[pl.ds(off, B)], dstv)
        pltpu.sync_copy(eep_hbm.at[pl.ds(off, B), :], valv)
        for j in range(B // 16):
            d = dstv[pl.ds(j * 16, 16)]
            ok = (d >= base) & (d < base + SC1_HALF)
            idxv[pl.ds(j * 16, 16)] = jnp.where(ok, d - base, SC1_HALF)
        pltpu.sync_copy(valv, accum.at[idxv], add=True)

    plsc.subcore_barrier()
    @pl.when(s < 15)
    def _():
        pltpu.sync_copy(
            accum.at[pl.ds(s * SC1_STRIPE, SC1_STRIPE), :],
            out_hbm.at[pl.ds(base + s * SC1_STRIPE, SC1_STRIPE), :])

    @pl.when(s == 15)
    def _():
        pltpu.sync_copy(
            accum.at[pl.ds(15 * SC1_STRIPE, SC1_HALF - 15 * SC1_STRIPE), :],
            out_hbm.at[pl.ds(base + 15 * SC1_STRIPE,
                             SC1_HALF - 15 * SC1_STRIPE), :])


def _sc1_call(eep, dstp, z1):
    return pl.kernel(
        _sc1_body,
        out_type=jax.ShapeDtypeStruct((N, SC1_W), jnp.float32),
        mesh=_SC_MESH,
        compiler_params=_SC_PARAMS,
        scratch_types=[
            pltpu.VMEM((B,), jnp.int32),
            pltpu.VMEM((B,), jnp.int32),
            pltpu.VMEM((B, SC1_W), jnp.float32),
            pltpu.VMEM_SHARED((SC1_ROWS, SC1_W), jnp.float32),
        ],
    )(eep, dstp, z1)


def _sc3_body(whel_hbm, dtab_hbm, src_hbm, dst_hbm, mx_hbm, z3_hbm, out_hbm,
              srcv0, srcv1, dstv0, dstv1, idxv0, idxv1, mxv,
              dbuf0, dbuf1, wbuf0, wbuf1, sbuf0, sbuf1,
              semA0, semA1, semB0, semB1, semS0, semS1, accum):
    c = lax.axis_index("c")
    s = lax.axis_index("s")
    tile_off = s * EDGES_PER_TILE
    pltpu.sync_copy(mx_hbm, mxv)
    mxvec = mxv[...]
    bufs = ((srcv0, dstv0, idxv0, dbuf0, wbuf0, sbuf0, semA0, semB0, semS0),
            (srcv1, dstv1, idxv1, dbuf1, wbuf1, sbuf1, semA1, semB1, semS1))

    for k in range(2):
        chunk = c * 2 + k
        base = chunk * SC3_CHUNK
        hi = jnp.minimum(base + SC3_CHUNK, N)
        pltpu.sync_copy(z3_hbm, accum.at[pl.ds(s * SC3_STRIPE, SC3_STRIPE), :])
        plsc.subcore_barrier()

        for b in range(2):
            srcv, dstv, idxv, dbuf, wbuf, sbuf, semA, semB, semS = bufs[b]
            off = tile_off + b * B3
            pltpu.sync_copy(src_hbm.at[pl.ds(off, B3)], srcv)
            pltpu.sync_copy(dst_hbm.at[pl.ds(off, B3)], dstv)
            pltpu.async_copy(whel_hbm.at[srcv], wbuf, semA)
            pltpu.async_copy(dtab_hbm.at[dstv], dbuf, semB)

        @pl.loop(0, NBLOCKS3_TILE, step=2)
        def _blk(g):
            for b in range(2):
                srcv, dstv, idxv, dbuf, wbuf, sbuf, semA, semB, semS = bufs[b]
                blk = g + b
                pltpu.make_async_copy(whel_hbm.at[srcv], wbuf, semA).wait()
                pltpu.make_async_copy(dtab_hbm.at[dstv], dbuf, semB).wait()

                @pl.when(blk >= 2)
                def _():
                    pltpu.make_async_copy(
                        sbuf, accum.at[idxv], semS).wait()

                @pl.loop(0, B3, unroll=4)
                def _edge(j):
                    elv = wbuf[j, pl.ds(H1, 16)]
                    erv = dbuf[j, pl.ds(0, 16)]
                    e_ = _lrelu(elv + erv)
                    M_ = _lrelu(mxvec + erv)
                    wv = jnp.exp(e_ - M_)
                    sbuf[j, pl.ds(H1, 16)] = wv
                    for hh in range(H):
                        bv = jnp.broadcast_to(wv[hh], (16,))
                        for t in range(2):
                            col = hh * 32 + t * 16
                            sbuf[j, pl.ds(col, 16)] = bv * wbuf[j, pl.ds(col, 16)]

                d = dstv[pl.ds(0, 16)]
                ok = (d >= base) & (d < hi)
                idxv[pl.ds(0, 16)] = jnp.where(ok, d - base, SC3_CHUNK)
                pltpu.async_copy(sbuf, accum.at[idxv], semS, add=True)

                nxt = blk + 2

                @pl.when(nxt < NBLOCKS3_TILE)
                def _():
                    off2 = tile_off + nxt * B3
                    pltpu.sync_copy(src_hbm.at[pl.ds(off2, B3)], srcv)
                    pltpu.sync_copy(dst_hbm.at[pl.ds(off2, B3)], dstv)
                    pltpu.async_copy(whel_hbm.at[srcv], wbuf, semA)
                    pltpu.async_copy(dtab_hbm.at[dstv], dbuf, semB)

        for b in range(2):
            srcv, dstv, idxv, dbuf, wbuf, sbuf, semA, semB, semS = bufs[b]
            pltpu.make_async_copy(sbuf, accum.at[idxv], semS).wait()

        plsc.subcore_barrier()

        @pl.when(s < 15)
        def _():
            pltpu.sync_copy(
                accum.at[pl.ds(s * SC3_STRIPE, SC3_STRIPE), :],
                out_hbm.at[pl.ds(base + s * SC3_STRIPE, SC3_STRIPE), :])

        full15 = 15 * SC3_STRIPE
        if k == 0:
            @pl.when(s == 15)
            def _():
                pltpu.sync_copy(
                    accum.at[pl.ds(full15, SC3_CHUNK - full15), :],
                    out_hbm.at[pl.ds(base + full15, SC3_CHUNK - full15), :])
        else:
            @pl.when((s == 15) & (c == 0))
            def _():
                pltpu.sync_copy(
                    accum.at[pl.ds(full15, SC3_CHUNK - full15), :],
                    out_hbm.at[pl.ds(base + full15, SC3_CHUNK - full15), :])

            @pl.when((s == 15) & (c == 1))
            def _():
                pltpu.sync_copy(
                    accum.at[pl.ds(full15, N - 3 * SC3_CHUNK - full15), :],
                    out_hbm.at[pl.ds(base + full15,
                                     N - 3 * SC3_CHUNK - full15), :])

        plsc.subcore_barrier()


def _sc3_call(whel, dtab, srcp, dstp, mx16, z3):
    return pl.kernel(
        _sc3_body,
        out_type=jax.ShapeDtypeStruct((N, SC3_W), jnp.float32),
        mesh=_SC_MESH,
        compiler_params=_SC_PARAMS,
        scratch_types=[
            pltpu.VMEM((B3,), jnp.int32),
            pltpu.VMEM((B3,), jnp.int32),
            pltpu.VMEM((B3,), jnp.int32),
            pltpu.VMEM((B3,), jnp.int32),
            pltpu.VMEM((B3,), jnp.int32),
            pltpu.VMEM((B3,), jnp.int32),
            pltpu.VMEM((16,), jnp.float32),
            pltpu.VMEM((B3, 16), jnp.float32),
            pltpu.VMEM((B3, 16), jnp.float32),
            pltpu.VMEM((B3, SC3_W), jnp.float32),
            pltpu.VMEM((B3, SC3_W), jnp.float32),
            pltpu.VMEM((B3, SC3_W), jnp.float32),
            pltpu.VMEM((B3, SC3_W), jnp.float32),
            pltpu.SemaphoreType.DMA,
            pltpu.SemaphoreType.DMA,
            pltpu.SemaphoreType.DMA,
            pltpu.SemaphoreType.DMA,
            pltpu.SemaphoreType.DMA,
            pltpu.SemaphoreType.DMA,
            pltpu.VMEM_SHARED((SC3_ROWS, SC3_W), jnp.float32),
        ],
    )(whel, dtab, srcp, dstp, mx16, z3)


def kernel(x, edge_index, edge_attr, Wf, bf, gf, betaf, We, be, ge, betae,
           Wfc, al, ar, bgat, g1, beta1, Wgate, bgate, Wo, bo, go, betao):
    f32 = jnp.float32
    xp = jnp.zeros((NP, 8), f32).at[:N, :7].set(x)
    Wf8 = jnp.zeros((8, H0), f32).at[:7].set(Wf)
    eap = jnp.zeros((EP, 8), f32).at[:E, :6].set(edge_attr)
    We8 = jnp.zeros((8, H0), f32).at[:6].set(We)
    srcp = jnp.concatenate([edge_index[0], jnp.zeros((EP - E,), jnp.int32)])
    dstp = jnp.concatenate([edge_index[1],
                            jnp.full((EP - E,), N, jnp.int32)])
    Al = jnp.zeros((H1, H), f32)
    Ar = jnp.zeros((H1, H), f32)
    for hh in range(H):
        Al = Al.at[hh * DH:(hh + 1) * DH, hh].set(al[hh])
        Ar = Ar.at[hh * DH:(hh + 1) * DH, hh].set(ar[hh])
    S4 = jnp.kron(jnp.eye(H, dtype=f32), jnp.ones((1, DH), f32))
    Wg8 = jnp.zeros((H1, 8), f32).at[:, 0:1].set(Wgate)
    bg8 = jnp.zeros((1, 8), f32).at[0, 0].set(bgate[0])
    r = lambda v: v.reshape(1, -1)
    z1 = jnp.zeros((SC1_STRIPE, SC1_W), f32)
    z3 = jnp.zeros((SC3_STRIPE, SC3_W), f32)

    h = pl.pallas_call(
        _tc_face_body,
        grid=(NBLK,),
        in_specs=[
            pl.BlockSpec((RB, 8), lambda i: (i, 0)),
            pl.BlockSpec((8, H0), lambda i: (0, 0)),
            _row_spec(H0), _row_spec(H0), _row_spec(H0),
        ],
        out_specs=pl.BlockSpec((RB, H0), lambda i: (i, 0)),
        out_shape=jax.ShapeDtypeStruct((NP, H0), f32),
    )(xp, Wf8, r(bf), r(gf), r(betaf))

    eep = pl.pallas_call(
        _tc_edge_body,
        grid=(EBLK,),
        in_specs=[
            pl.BlockSpec((EB, 8), lambda i: (i, 0)),
            pl.BlockSpec((8, H0), lambda i: (0, 0)),
            _row_spec(H0), _row_spec(H0), _row_spec(H0),
        ],
        out_specs=pl.BlockSpec((EB, SC1_W), lambda i: (i, 0)),
        out_shape=jax.ShapeDtypeStruct((EP, SC1_W), f32),
    )(eap, We8, r(be), r(ge), r(betae))

    ssc = _sc1_call(eep, dstp, z1)
    sscp = jnp.zeros((NP, SC1_W), f32).at[:N].set(ssc)

    whel, dtab, mx16 = pl.pallas_call(
        _tc_mid_body,
        grid=(NBLK,),
        in_specs=[
            pl.BlockSpec((RB, H0), lambda i: (i, 0)),
            pl.BlockSpec((RB, SC1_W), lambda i: (i, 0)),
            pl.BlockSpec((H0, H1), lambda i: (0, 0)),
            pl.BlockSpec((H1, H), lambda i: (0, 0)),
            pl.BlockSpec((H1, H), lambda i: (0, 0)),
            _row_spec(H0), _row_spec(H0), _row_spec(H0),
        ],
        out_specs=[
            pl.BlockSpec((RB, SC3_W), lambda i: (i, 0)),
            pl.BlockSpec((RB, 16), lambda i: (i, 0)),
            pl.BlockSpec((1, 16), lambda i: (0, 0)),
        ],
        out_shape=[
            jax.ShapeDtypeStruct((NP, SC3_W), f32),
            jax.ShapeDtypeStruct((NP, 16), f32),
            jax.ShapeDtypeStruct((1, 16), f32),
        ],
    )(h, sscp, Wfc, Al, Ar, r(be), r(ge), r(betae))

    nd = _sc3_call(whel, dtab, srcp, dstp, mx16.reshape(16), z3)
    ndp = jnp.zeros((NP, SC3_W), f32).at[:N].set(nd)

    nf, z8 = pl.pallas_call(
        _tc_post_body,
        grid=(NBLK,),
        in_specs=[
            pl.BlockSpec((RB, SC3_W), lambda i: (i, 0)),
            pl.BlockSpec((RB, SC3_W), lambda i: (i, 0)),
            pl.BlockSpec((RB, 16), lambda i: (i, 0)),
            pl.BlockSpec((1, 16), lambda i: (0, 0)),
            pl.BlockSpec((H, H1), lambda i: (0, 0)),
            _row_spec(H1), _row_spec(H1), _row_spec(H1),
            pl.BlockSpec((H1, 8), lambda i: (0, 0)),
            _row_spec(8),
        ],
        out_specs=[
            pl.BlockSpec((RB, H1), lambda i: (i, 0)),
            pl.BlockSpec((RB, 8), lambda i: (i, 0)),
        ],
        out_shape=[
            jax.ShapeDtypeStruct((NP, H1), f32),
            jax.ShapeDtypeStruct((NP, 8), f32),
        ],
    )(ndp, whel, dtab, mx16, S4, r(bgat), r(g1), r(beta1), Wg8, bg8)

    zmax8 = pl.pallas_call(
        _tc_zmax_body,
        grid=(NBLK,),
        in_specs=[pl.BlockSpec((RB, 8), lambda i: (i, 0))],
        out_specs=pl.BlockSpec((1, 8), lambda i: (0, 0)),
        out_shape=jax.ShapeDtypeStruct((1, 8), f32),
    )(z8)

    pooled, feat = pl.pallas_call(
        _tc_pool_body,
        grid=(NBLK,),
        in_specs=[
            pl.BlockSpec((RB, H1), lambda i: (i, 0)),
            pl.BlockSpec((RB, 8), lambda i: (i, 0)),
            pl.BlockSpec((1, 8), lambda i: (0, 0)),
            pl.BlockSpec((H1, H1), lambda i: (0, 0)),
            _row_spec(H1), _row_spec(H1), _row_spec(H1),
        ],
        out_specs=[
            pl.BlockSpec((1, H1), lambda i: (0, 0)),
            pl.BlockSpec((1, H1), lambda i: (0, 0)),
        ],
        out_shape=[
            jax.ShapeDtypeStruct((1, H1), f32),
            jax.ShapeDtypeStruct((1, H1), f32),
        ],
        scratch_shapes=[
            pltpu.VMEM((1, H1), f32),
            pltpu.VMEM((1, 8), f32),
        ],
    )(nf, z8, zmax8, Wo, r(bo), r(go), r(betao))

    return (feat, nf[:N], pooled)


# speedup vs baseline: 1.2130x; 1.2130x over previous
import functools

import jax
import jax.numpy as jnp
from jax import lax
from jax.experimental import pallas as pl
from jax.experimental.pallas import tpu as pltpu
from jax.experimental.pallas import tpu_sc as plsc

N = 50000
E = 800000
H = 4
DH = 32
H0 = 64
H1 = 128

NP = 50176
EP = 819200
NBLK = 98
RB = 512
EB = 1024
EBLK = EP // EB

SC1_HALF = 25000
SC1_ROWS = 25088
SC1_STRIPE = 1568
SC1_W = 72
SC3_CHUNK = 12512
SC3_ROWS = 12544
SC3_STRIPE = 784
SC3_W = 144
SC3_AW = 136
EDGES_PER_TILE = EP // 16
B = 128
NBLOCKS_TILE = EDGES_PER_TILE // B
B3 = 32
NBLOCKS3_TILE = EDGES_PER_TILE // B3


def _ln(h, g, b, eps=1e-5):
    mu = jnp.mean(h, axis=-1, keepdims=True)
    va = jnp.mean((h - mu) * (h - mu), axis=-1, keepdims=True)
    return (h - mu) / jnp.sqrt(va + eps) * g + b


def _lrelu(v):
    return jnp.maximum(v, 0.2 * v)


def _tc_face_body(xb, Wf8, bf, gf, betaf, hb):
    y = jax.nn.relu(jnp.dot(xb[...], Wf8[...],
                            preferred_element_type=jnp.float32) + bf[...])
    hb[...] = _ln(y, gf[...], betaf[...])


def _tc_edge_body(eab, We8, be, ge, betae, eepb):
    y = jax.nn.relu(jnp.dot(eab[...], We8[...],
                            preferred_element_type=jnp.float32) + be[...])
    ee = _ln(y, ge[...], betae[...])
    ones = jnp.ones((EB, 1), jnp.float32)
    zeros = jnp.zeros((EB, SC1_W - H0 - 1), jnp.float32)
    eepb[...] = jnp.concatenate([ee, ones, zeros], axis=1)


def _tc_mid_body(hb, sscb, Wfc, Al, Ar, be, ge, betae, whelb, dtabb, mx16b):
    i = pl.program_id(0)
    c0 = _ln(jax.nn.relu(be[...]), ge[...], betae[...])
    ssum = sscb[:, :H0]
    cnt = sscb[:, H0:H0 + 1]
    h2 = hb[...] + (ssum + c0) / (cnt + 1.0)
    Wh = jnp.dot(h2, Wfc[...], preferred_element_type=jnp.float32)
    el = jnp.dot(Wh, Al[...], preferred_element_type=jnp.float32)
    er = jnp.dot(Wh, Ar[...], preferred_element_type=jnp.float32)
    z8 = jnp.zeros((RB, 8), jnp.float32)
    z4 = jnp.zeros((RB, 4), jnp.float32)
    whelb[...] = jnp.concatenate([Wh, z8, el, z4], axis=1)
    dtabb[...] = jnp.concatenate([z8, er, z4], axis=1)
    elz = jnp.concatenate([z8, el, z4], axis=1)
    bm = jnp.max(elz, axis=0, keepdims=True)

    @pl.when(i == 0)
    def _():
        mx16b[...] = jnp.full((1, 16), -1e30, jnp.float32)

    mx16b[...] = jnp.maximum(mx16b[...], bm)


def _tc_post_body(ndb, whelb, dtabb, mx16b, S4, bgat, g1, beta1, Wg8, bg8,
                  nfb, z8b):
    Wh = whelb[:, :H1]
    elv = whelb[:, H1 + 8:H1 + 12]
    erv = dtabb[:, 8:12]
    mx = mx16b[:, 8:12]
    sE = _lrelu(elv + erv)
    M = _lrelu(mx + erv)
    sw = jnp.exp(sE - M)
    num = ndb[:, :H1] + jnp.dot(sw, S4[...],
                                preferred_element_type=jnp.float32) * Wh
    den = ndb[:, H1:H1 + H] + sw
    out = num / (jnp.dot(den, S4[...],
                         preferred_element_type=jnp.float32) + 1e-9)
    out = jax.nn.relu(out + bgat[...])
    nf = _ln(out, g1[...], beta1[...])
    nfb[...] = nf
    z8b[...] = jnp.dot(nf, Wg8[...], preferred_element_type=jnp.float32) + bg8[...]


def _tc_zmax_body(z8b, zmaxb):
    i = pl.program_id(0)
    rows = lax.broadcasted_iota(jnp.int32, (RB, 1), 0) + i * RB
    zm = jnp.where(rows < N, z8b[:, 0:1], -1e30)
    bm = jnp.max(zm)

    @pl.when(i == 0)
    def _():
        zmaxb[...] = jnp.full((1, 8), -1e30, jnp.float32)

    zmaxb[...] = jnp.maximum(zmaxb[...], bm)


def _tc_pool_body(nfb, z8b, zmaxb, Wo, bo, go, betao, pooledb, featb,
                  Sv, S1):
    i = pl.program_id(0)

    @pl.when(i == 0)
    def _():
        Sv[...] = jnp.zeros((1, H1), jnp.float32)
        S1[...] = jnp.zeros((1, 8), jnp.float32)

    rows = lax.broadcasted_iota(jnp.int32, (RB, 1), 0) + i * RB
    ge_ = jnp.where(rows < N, jnp.exp(z8b[:, 0:1] - zmaxb[:, 0:1]), 0.0)
    Sv[...] += jnp.sum(ge_ * nfb[...], axis=0, keepdims=True)
    S1[...] += jnp.full((1, 8), jnp.sum(ge_), jnp.float32)

    @pl.when(i == NBLK - 1)
    def _():
        pooled = Sv[...] / S1[:, 0:1]
        feat = jax.nn.relu(jnp.dot(pooled, Wo[...],
                                   preferred_element_type=jnp.float32) + bo[...])
        featb[...] = _ln(feat, go[...], betao[...])
        pooledb[...] = pooled


def _row_spec(w):
    return pl.BlockSpec((1, w), lambda i: (0, 0))


_SC_MESH = plsc.VectorSubcoreMesh(core_axis_name="c", subcore_axis_name="s")
_SC_PARAMS = pltpu.CompilerParams(use_tc_tiling_on_sc=False)


def _sc1_body(eep_hbm, dst_hbm, z1_hbm, out_hbm, dstv, idxv, valv, accum):
    c = lax.axis_index("c")
    s = lax.axis_index("s")
    base = c * SC1_HALF
    pltpu.sync_copy(z1_hbm, accum.at[pl.ds(s * SC1_STRIPE, SC1_STRIPE), :])
    plsc.subcore_barrier()

    tile_off = s * EDGES_PER_TILE

    @pl.loop(0, NBLOCKS_TILE)
    def _blk(blk):
        off = tile_off + blk * B
        pltpu.sync_copy(dst_hbm.at[pl.ds(off, B)], dstv)
        pltpu.sync_copy(eep_hbm.at[pl.ds(off, B), :], valv)
        for j in range(B // 16):
            d = dstv[pl.ds(j * 16, 16)]
            ok = (d >= base) & (d < base + SC1_HALF)
            idxv[pl.ds(j * 16, 16)] = jnp.where(ok, d - base, SC1_HALF)
        pltpu.sync_copy(valv, accum.at[idxv], add=True)

    plsc.subcore_barrier()
    @pl.when(s < 15)
    def _():
        pltpu.sync_copy(
            accum.at[pl.ds(s * SC1_STRIPE, SC1_STRIPE), :],
            out_hbm.at[pl.ds(base + s * SC1_STRIPE, SC1_STRIPE), :])

    @pl.when(s == 15)
    def _():
        pltpu.sync_copy(
            accum.at[pl.ds(15 * SC1_STRIPE, SC1_HALF - 15 * SC1_STRIPE), :],
            out_hbm.at[pl.ds(base + 15 * SC1_STRIPE,
                             SC1_HALF - 15 * SC1_STRIPE), :])


def _sc1_call(eep, dstp, z1):
    return pl.kernel(
        _sc1_body,
        out_type=jax.ShapeDtypeStruct((N, SC1_W), jnp.float32),
        mesh=_SC_MESH,
        compiler_params=_SC_PARAMS,
        scratch_types=[
            pltpu.VMEM((B,), jnp.int32),
            pltpu.VMEM((B,), jnp.int32),
            pltpu.VMEM((B, SC1_W), jnp.float32),
            pltpu.VMEM_SHARED((SC1_ROWS, SC1_W), jnp.float32),
        ],
    )(eep, dstp, z1)


def _sc3_body(whel_hbm, dtab_hbm, src_hbm, dst_hbm, mx_hbm, z3_hbm, out_hbm,
              srcv0, srcv1, dstv0, dstv1, idxv0, idxv1, mxv,
              dbuf0, dbuf1, wbuf0, wbuf1, sbuf0, sbuf1,
              semA0, semA1, semB0, semB1, semS0, semS1, accum):
    c = lax.axis_index("c")
    s = lax.axis_index("s")
    tile_off = s * EDGES_PER_TILE
    pltpu.sync_copy(mx_hbm, mxv)
    mxvec = mxv[...]
    bufs = ((srcv0, dstv0, idxv0, dbuf0, wbuf0, sbuf0, semA0, semB0, semS0),
            (srcv1, dstv1, idxv1, dbuf1, wbuf1, sbuf1, semA1, semB1, semS1))

    for k in range(2):
        chunk = c * 2 + k
        base = chunk * SC3_CHUNK
        hi = jnp.minimum(base + SC3_CHUNK, N)
        pltpu.sync_copy(z3_hbm, accum.at[pl.ds(s * SC3_STRIPE, SC3_STRIPE), :])
        plsc.subcore_barrier()

        for b in range(2):
            srcv, dstv, idxv, dbuf, wbuf, sbuf, semA, semB, semS = bufs[b]
            off = tile_off + b * B3
            pltpu.sync_copy(src_hbm.at[pl.ds(off, B3)], srcv)
            pltpu.sync_copy(dst_hbm.at[pl.ds(off, B3)], dstv)
            pltpu.async_copy(whel_hbm.at[srcv], wbuf, semA)
            pltpu.async_copy(dtab_hbm.at[dstv], dbuf, semB)

        @pl.loop(0, NBLOCKS3_TILE, step=2)
        def _blk(g):
            for b in range(2):
                srcv, dstv, idxv, dbuf, wbuf, sbuf, semA, semB, semS = bufs[b]
                blk = g + b
                pltpu.make_async_copy(whel_hbm.at[srcv], wbuf, semA).wait()
                pltpu.make_async_copy(dtab_hbm.at[dstv], dbuf, semB).wait()

                @pl.when(blk >= 2)
                def _():
                    pltpu.make_async_copy(
                        sbuf, accum.at[idxv], semS).wait()

                @pl.loop(0, B3, unroll=4)
                def _edge(j):
                    elv = wbuf[j, pl.ds(H1, 16)]
                    erv = dbuf[j, pl.ds(0, 16)]
                    e_ = _lrelu(elv + erv)
                    M_ = _lrelu(mxvec + erv)
                    wv = jnp.exp(e_ - M_)
                    sbuf[j, pl.ds(120, 16)] = wv
                    for hh in range(H):
                        bv = jnp.broadcast_to(wv[8 + hh], (16,))
                        for t in range(2):
                            col = hh * 32 + t * 16
                            sbuf[j, pl.ds(col, 16)] = bv * wbuf[j, pl.ds(col, 16)]

                for jj in range(B3 // 16):
                    d = dstv[pl.ds(jj * 16, 16)]
                    ok = (d >= base) & (d < hi)
                    idxv[pl.ds(jj * 16, 16)] = jnp.where(ok, d - base, SC3_CHUNK)
                pltpu.async_copy(sbuf, accum.at[idxv], semS, add=True)

                nxt = blk + 2

                @pl.when(nxt < NBLOCKS3_TILE)
                def _():
                    off2 = tile_off + nxt * B3
                    pltpu.sync_copy(src_hbm.at[pl.ds(off2, B3)], srcv)
                    pltpu.sync_copy(dst_hbm.at[pl.ds(off2, B3)], dstv)
                    pltpu.async_copy(whel_hbm.at[srcv], wbuf, semA)
                    pltpu.async_copy(dtab_hbm.at[dstv], dbuf, semB)

        for b in range(2):
            srcv, dstv, idxv, dbuf, wbuf, sbuf, semA, semB, semS = bufs[b]
            pltpu.make_async_copy(sbuf, accum.at[idxv], semS).wait()

        plsc.subcore_barrier()

        @pl.when(s < 15)
        def _():
            pltpu.sync_copy(
                accum.at[pl.ds(s * SC3_STRIPE, SC3_STRIPE), :],
                out_hbm.at[pl.ds(base + s * SC3_STRIPE, SC3_STRIPE), :])

        full15 = 15 * SC3_STRIPE
        if k == 0:
            @pl.when(s == 15)
            def _():
                pltpu.sync_copy(
                    accum.at[pl.ds(full15, SC3_CHUNK - full15), :],
                    out_hbm.at[pl.ds(base + full15, SC3_CHUNK - full15), :])
        else:
            @pl.when((s == 15) & (c == 0))
            def _():
                pltpu.sync_copy(
                    accum.at[pl.ds(full15, SC3_CHUNK - full15), :],
                    out_hbm.at[pl.ds(base + full15, SC3_CHUNK - full15), :])

            @pl.when((s == 15) & (c == 1))
            def _():
                pltpu.sync_copy(
                    accum.at[pl.ds(full15, N - 3 * SC3_CHUNK - full15), :],
                    out_hbm.at[pl.ds(base + full15,
                                     N - 3 * SC3_CHUNK - full15), :])

        plsc.subcore_barrier()


def _sc3_call(whel, dtab, srcp, dstp, mx16, z3):
    return pl.kernel(
        _sc3_body,
        out_type=jax.ShapeDtypeStruct((N, SC3_AW), jnp.float32),
        mesh=_SC_MESH,
        compiler_params=_SC_PARAMS,
        scratch_types=[
            pltpu.VMEM((B3,), jnp.int32),
            pltpu.VMEM((B3,), jnp.int32),
            pltpu.VMEM((B3,), jnp.int32),
            pltpu.VMEM((B3,), jnp.int32),
            pltpu.VMEM((B3,), jnp.int32),
            pltpu.VMEM((B3,), jnp.int32),
            pltpu.VMEM((16,), jnp.float32),
            pltpu.VMEM((B3, 16), jnp.float32),
            pltpu.VMEM((B3, 16), jnp.float32),
            pltpu.VMEM((B3, SC3_W), jnp.float32),
            pltpu.VMEM((B3, SC3_W), jnp.float32),
            pltpu.VMEM((B3, SC3_AW), jnp.float32),
            pltpu.VMEM((B3, SC3_AW), jnp.float32),
            pltpu.SemaphoreType.DMA,
            pltpu.SemaphoreType.DMA,
            pltpu.SemaphoreType.DMA,
            pltpu.SemaphoreType.DMA,
            pltpu.SemaphoreType.DMA,
            pltpu.SemaphoreType.DMA,
            pltpu.VMEM_SHARED((SC3_ROWS, SC3_AW), jnp.float32),
        ],
    )(whel, dtab, srcp, dstp, mx16, z3)


def kernel(x, edge_index, edge_attr, Wf, bf, gf, betaf, We, be, ge, betae,
           Wfc, al, ar, bgat, g1, beta1, Wgate, bgate, Wo, bo, go, betao):
    f32 = jnp.float32
    xp = jnp.zeros((NP, 8), f32).at[:N, :7].set(x)
    Wf8 = jnp.zeros((8, H0), f32).at[:7].set(Wf)
    eap = jnp.zeros((EP, 8), f32).at[:E, :6].set(edge_attr)
    We8 = jnp.zeros((8, H0), f32).at[:6].set(We)
    srcp = jnp.concatenate([edge_index[0], jnp.zeros((EP - E,), jnp.int32)])
    dstp = jnp.concatenate([edge_index[1],
                            jnp.full((EP - E,), N, jnp.int32)])
    Al = jnp.zeros((H1, H), f32)
    Ar = jnp.zeros((H1, H), f32)
    for hh in range(H):
        Al = Al.at[hh * DH:(hh + 1) * DH, hh].set(al[hh])
        Ar = Ar.at[hh * DH:(hh + 1) * DH, hh].set(ar[hh])
    S4 = jnp.kron(jnp.eye(H, dtype=f32), jnp.ones((1, DH), f32))
    Wg8 = jnp.zeros((H1, 8), f32).at[:, 0:1].set(Wgate)
    bg8 = jnp.zeros((1, 8), f32).at[0, 0].set(bgate[0])
    r = lambda v: v.reshape(1, -1)
    z1 = jnp.zeros((SC1_STRIPE, SC1_W), f32)
    z3 = jnp.zeros((SC3_STRIPE, SC3_AW), f32)

    h = pl.pallas_call(
        _tc_face_body,
        grid=(NBLK,),
        in_specs=[
            pl.BlockSpec((RB, 8), lambda i: (i, 0)),
            pl.BlockSpec((8, H0), lambda i: (0, 0)),
            _row_spec(H0), _row_spec(H0), _row_spec(H0),
        ],
        out_specs=pl.BlockSpec((RB, H0), lambda i: (i, 0)),
        out_shape=jax.ShapeDtypeStruct((NP, H0), f32),
    )(xp, Wf8, r(bf), r(gf), r(betaf))

    eep = pl.pallas_call(
        _tc_edge_body,
        grid=(EBLK,),
        in_specs=[
            pl.BlockSpec((EB, 8), lambda i: (i, 0)),
            pl.BlockSpec((8, H0), lambda i: (0, 0)),
            _row_spec(H0), _row_spec(H0), _row_spec(H0),
        ],
        out_specs=pl.BlockSpec((EB, SC1_W), lambda i: (i, 0)),
        out_shape=jax.ShapeDtypeStruct((EP, SC1_W), f32),
    )(eap, We8, r(be), r(ge), r(betae))

    ssc = _sc1_call(eep, dstp, z1)
    sscp = jnp.zeros((NP, SC1_W), f32).at[:N].set(ssc)

    whel, dtab, mx16 = pl.pallas_call(
        _tc_mid_body,
        grid=(NBLK,),
        in_specs=[
            pl.BlockSpec((RB, H0), lambda i: (i, 0)),
            pl.BlockSpec((RB, SC1_W), lambda i: (i, 0)),
            pl.BlockSpec((H0, H1), lambda i: (0, 0)),
            pl.BlockSpec((H1, H), lambda i: (0, 0)),
            pl.BlockSpec((H1, H), lambda i: (0, 0)),
            _row_spec(H0), _row_spec(H0), _row_spec(H0),
        ],
        out_specs=[
            pl.BlockSpec((RB, SC3_W), lambda i: (i, 0)),
            pl.BlockSpec((RB, 16), lambda i: (i, 0)),
            pl.BlockSpec((1, 16), lambda i: (0, 0)),
        ],
        out_shape=[
            jax.ShapeDtypeStruct((NP, SC3_W), f32),
            jax.ShapeDtypeStruct((NP, 16), f32),
            jax.ShapeDtypeStruct((1, 16), f32),
        ],
    )(h, sscp, Wfc, Al, Ar, r(be), r(ge), r(betae))

    nd = _sc3_call(whel, dtab, srcp, dstp, mx16.reshape(16), z3)
    ndp = jnp.zeros((NP, SC3_AW), f32).at[:N].set(nd)

    nf, z8 = pl.pallas_call(
        _tc_post_body,
        grid=(NBLK,),
        in_specs=[
            pl.BlockSpec((RB, SC3_AW), lambda i: (i, 0)),
            pl.BlockSpec((RB, SC3_W), lambda i: (i, 0)),
            pl.BlockSpec((RB, 16), lambda i: (i, 0)),
            pl.BlockSpec((1, 16), lambda i: (0, 0)),
            pl.BlockSpec((H, H1), lambda i: (0, 0)),
            _row_spec(H1), _row_spec(H1), _row_spec(H1),
            pl.BlockSpec((H1, 8), lambda i: (0, 0)),
            _row_spec(8),
        ],
        out_specs=[
            pl.BlockSpec((RB, H1), lambda i: (i, 0)),
            pl.BlockSpec((RB, 8), lambda i: (i, 0)),
        ],
        out_shape=[
            jax.ShapeDtypeStruct((NP, H1), f32),
            jax.ShapeDtypeStruct((NP, 8), f32),
        ],
    )(ndp, whel, dtab, mx16, S4, r(bgat), r(g1), r(beta1), Wg8, bg8)

    zmax8 = pl.pallas_call(
        _tc_zmax_body,
        grid=(NBLK,),
        in_specs=[pl.BlockSpec((RB, 8), lambda i: (i, 0))],
        out_specs=pl.BlockSpec((1, 8), lambda i: (0, 0)),
        out_shape=jax.ShapeDtypeStruct((1, 8), f32),
    )(z8)

    pooled, feat = pl.pallas_call(
        _tc_pool_body,
        grid=(NBLK,),
        in_specs=[
            pl.BlockSpec((RB, H1), lambda i: (i, 0)),
            pl.BlockSpec((RB, 8), lambda i: (i, 0)),
            pl.BlockSpec((1, 8), lambda i: (0, 0)),
            pl.BlockSpec((H1, H1), lambda i: (0, 0)),
            _row_spec(H1), _row_spec(H1), _row_spec(H1),
        ],
        out_specs=[
            pl.BlockSpec((1, H1), lambda i: (0, 0)),
            pl.BlockSpec((1, H1), lambda i: (0, 0)),
        ],
        out_shape=[
            jax.ShapeDtypeStruct((1, H1), f32),
            jax.ShapeDtypeStruct((1, H1), f32),
        ],
        scratch_shapes=[
            pltpu.VMEM((1, H1), f32),
            pltpu.VMEM((1, 8), f32),
        ],
    )(nf, z8, zmax8, Wo, r(bo), r(go), r(betao))

    return (feat, nf[:N], pooled)


# speedup vs baseline: 1.5362x; 1.2664x over previous
import functools

import jax
import jax.numpy as jnp
from jax import lax
from jax.experimental import pallas as pl
from jax.experimental.pallas import tpu as pltpu
from jax.experimental.pallas import tpu_sc as plsc

N = 50000
E = 800000
H = 4
DH = 32
H0 = 64
H1 = 128

NP = 50176
EP = 819200
NBLK = 98
RB = 512
EB = 1024
EBLK = EP // EB

SC1_HALF = 25000
SC1_ROWS = 25088
SC1_STRIPE = 1568
SC1_W = 72
SC3_CHUNK = 12512
SC3_ROWS = 12544
SC3_STRIPE = 784
SC3_W = 136
SC3_AW = 136
EDGES_PER_TILE = EP // 16
B = 128
NBLOCKS_TILE = EDGES_PER_TILE // B
B3 = 64
NBLOCKS3_TILE = EDGES_PER_TILE // B3


def _ln(h, g, b, eps=1e-5):
    mu = jnp.mean(h, axis=-1, keepdims=True)
    va = jnp.mean((h - mu) * (h - mu), axis=-1, keepdims=True)
    return (h - mu) / jnp.sqrt(va + eps) * g + b


def _lrelu(v):
    return jnp.maximum(v, 0.2 * v)


def _tc_face_body(xb, Wf8, bf, gf, betaf, hb):
    y = jax.nn.relu(jnp.dot(xb[...], Wf8[...],
                            preferred_element_type=jnp.float32) + bf[...])
    hb[...] = _ln(y, gf[...], betaf[...])


def _tc_edge_body(eab, We8, be, ge, betae, eepb):
    y = jax.nn.relu(jnp.dot(eab[...], We8[...],
                            preferred_element_type=jnp.float32) + be[...])
    ee = _ln(y, ge[...], betae[...])
    ones = jnp.ones((EB, 1), jnp.float32)
    zeros = jnp.zeros((EB, SC1_W - H0 - 1), jnp.float32)
    eepb[...] = jnp.concatenate([ee, ones, zeros], axis=1)


def _tc_mid_body(hb, sscb, Wfc, Al, Ar, be, ge, betae, whelb, dtabb, mx16b):
    i = pl.program_id(0)
    c0 = _ln(jax.nn.relu(be[...]), ge[...], betae[...])
    ssum = sscb[:, :H0]
    cnt = sscb[:, H0:H0 + 1]
    h2 = hb[...] + (ssum + c0) / (cnt + 1.0)
    Wh = jnp.dot(h2, Wfc[...], preferred_element_type=jnp.float32)
    el = jnp.dot(Wh, Al[...], preferred_element_type=jnp.float32)
    er = jnp.dot(Wh, Ar[...], preferred_element_type=jnp.float32)
    z8 = jnp.zeros((RB, 8), jnp.float32)
    z4 = jnp.zeros((RB, 4), jnp.float32)
    whelb[...] = jnp.concatenate([Wh, el, z4], axis=1)
    dtabb[...] = jnp.concatenate([z8, er, z4], axis=1)
    elz = jnp.concatenate([z8, el, z4], axis=1)
    bm = jnp.max(elz, axis=0, keepdims=True)

    @pl.when(i == 0)
    def _():
        mx16b[...] = jnp.full((1, 16), -1e30, jnp.float32)

    mx16b[...] = jnp.maximum(mx16b[...], bm)


def _tc_post_body(ndb, whelb, dtabb, mx16b, S4, bgat, g1, beta1, Wg8, bg8,
                  nfb, z8b):
    Wh = whelb[:, :H1]
    elv = whelb[:, H1:H1 + H]
    erv = dtabb[:, 8:12]
    mx = mx16b[:, 8:12]
    sE = _lrelu(elv + erv)
    M = _lrelu(mx + erv)
    sw = jnp.exp(sE - M)
    num = ndb[:, :H1] + jnp.dot(sw, S4[...],
                                preferred_element_type=jnp.float32) * Wh
    den = ndb[:, H1:H1 + H] + sw
    out = num / (jnp.dot(den, S4[...],
                         preferred_element_type=jnp.float32) + 1e-9)
    out = jax.nn.relu(out + bgat[...])
    nf = _ln(out, g1[...], beta1[...])
    nfb[...] = nf
    z8b[...] = jnp.dot(nf, Wg8[...], preferred_element_type=jnp.float32) + bg8[...]


def _tc_zmax_body(z8b, zmaxb):
    i = pl.program_id(0)
    rows = lax.broadcasted_iota(jnp.int32, (RB, 1), 0) + i * RB
    zm = jnp.where(rows < N, z8b[:, 0:1], -1e30)
    bm = jnp.max(zm)

    @pl.when(i == 0)
    def _():
        zmaxb[...] = jnp.full((1, 8), -1e30, jnp.float32)

    zmaxb[...] = jnp.maximum(zmaxb[...], bm)


def _tc_pool_body(nfb, z8b, zmaxb, Wo, bo, go, betao, pooledb, featb,
                  Sv, S1):
    i = pl.program_id(0)

    @pl.when(i == 0)
    def _():
        Sv[...] = jnp.zeros((1, H1), jnp.float32)
        S1[...] = jnp.zeros((1, 8), jnp.float32)

    rows = lax.broadcasted_iota(jnp.int32, (RB, 1), 0) + i * RB
    ge_ = jnp.where(rows < N, jnp.exp(z8b[:, 0:1] - zmaxb[:, 0:1]), 0.0)
    Sv[...] += jnp.sum(ge_ * nfb[...], axis=0, keepdims=True)
    S1[...] += jnp.full((1, 8), jnp.sum(ge_), jnp.float32)

    @pl.when(i == NBLK - 1)
    def _():
        pooled = Sv[...] / S1[:, 0:1]
        feat = jax.nn.relu(jnp.dot(pooled, Wo[...],
                                   preferred_element_type=jnp.float32) + bo[...])
        featb[...] = _ln(feat, go[...], betao[...])
        pooledb[...] = pooled


def _row_spec(w):
    return pl.BlockSpec((1, w), lambda i: (0, 0))


_SC_MESH = plsc.VectorSubcoreMesh(core_axis_name="c", subcore_axis_name="s")
_SC_PARAMS = pltpu.CompilerParams(use_tc_tiling_on_sc=False)


def _sc1_body(eep_hbm, dst_hbm, z1_hbm, out_hbm, dstv, idxv, valv, accum):
    c = lax.axis_index("c")
    s = lax.axis_index("s")
    base = c * SC1_HALF
    pltpu.sync_copy(z1_hbm, accum.at[pl.ds(s * SC1_STRIPE, SC1_STRIPE), :])
    plsc.subcore_barrier()

    tile_off = s * EDGES_PER_TILE

    @pl.loop(0, NBLOCKS_TILE)
    def _blk(blk):
        off = tile_off + blk * B
        pltpu.sync_copy(dst_hbm.at[pl.ds(off, B)], dstv)
        pltpu.sync_copy(eep_hbm.at[pl.ds(off, B), :], valv)
        for j in range(B // 16):
            d = dstv[pl.ds(j * 16, 16)]
            ok = (d >= base) & (d < base + SC1_HALF)
            idxv[pl.ds(j * 16, 16)] = jnp.where(ok, d - base, SC1_HALF)
        pltpu.sync_copy(valv, accum.at[idxv], add=True)

    plsc.subcore_barrier()
    @pl.when(s < 15)
    def _():
        pltpu.sync_copy(
            accum.at[pl.ds(s * SC1_STRIPE, SC1_STRIPE), :],
            out_hbm.at[pl.ds(base + s * SC1_STRIPE, SC1_STRIPE), :])

    @pl.when(s == 15)
    def _():
        pltpu.sync_copy(
            accum.at[pl.ds(15 * SC1_STRIPE, SC1_HALF - 15 * SC1_STRIPE), :],
            out_hbm.at[pl.ds(base + 15 * SC1_STRIPE,
                             SC1_HALF - 15 * SC1_STRIPE), :])


def _sc1_call(eep, dstp, z1):
    return pl.kernel(
        _sc1_body,
        out_type=jax.ShapeDtypeStruct((N, SC1_W), jnp.float32),
        mesh=_SC_MESH,
        compiler_params=_SC_PARAMS,
        scratch_types=[
            pltpu.VMEM((B,), jnp.int32),
            pltpu.VMEM((B,), jnp.int32),
            pltpu.VMEM((B, SC1_W), jnp.float32),
            pltpu.VMEM_SHARED((SC1_ROWS, SC1_W), jnp.float32),
        ],
    )(eep, dstp, z1)


def _sc3_body(whel_hbm, dtab_hbm, src_hbm, dst_hbm, mx_hbm, z3_hbm, out_hbm,
              srcv0, srcv1, dstv0, dstv1, idxv0, idxv1, mxv,
              dbuf0, dbuf1, wbuf0, wbuf1,
              semA0, semA1, semB0, semB1, accum):
    c = lax.axis_index("c")
    s = lax.axis_index("s")
    tile_off = s * EDGES_PER_TILE
    pltpu.sync_copy(mx_hbm, mxv)
    mxvec = mxv[...]
    bufs = ((srcv0, dstv0, idxv0, dbuf0, wbuf0, semA0, semB0),
            (srcv1, dstv1, idxv1, dbuf1, wbuf1, semA1, semB1))

    for k in range(2):
        chunk = c * 2 + k
        base = chunk * SC3_CHUNK
        hi = jnp.minimum(base + SC3_CHUNK, N)
        pltpu.sync_copy(z3_hbm, accum.at[pl.ds(s * SC3_STRIPE, SC3_STRIPE), :])
        plsc.subcore_barrier()

        for b in range(2):
            srcv, dstv, idxv, dbuf, wbuf, semA, semB = bufs[b]
            off = tile_off + b * B3
            pltpu.sync_copy(src_hbm.at[pl.ds(off, B3)], srcv)
            pltpu.sync_copy(dst_hbm.at[pl.ds(off, B3)], dstv)
            pltpu.async_copy(whel_hbm.at[srcv], wbuf, semA)
            pltpu.async_copy(dtab_hbm.at[dstv], dbuf, semB)

        @pl.loop(0, NBLOCKS3_TILE, step=2)
        def _blk(g):
            for b in range(2):
                srcv, dstv, idxv, dbuf, wbuf, semA, semB = bufs[b]
                blk = g + b
                pltpu.make_async_copy(whel_hbm.at[srcv], wbuf, semA).wait()
                pltpu.make_async_copy(dtab_hbm.at[dstv], dbuf, semB).wait()

                lane = lax.iota(jnp.int32, 16)
                lo8 = lane < 8

                @pl.loop(0, B3, unroll=4)
                def _edge(j):
                    elv = wbuf[j, pl.ds(120, 16)]
                    erv = dbuf[j, pl.ds(0, 16)]
                    e_ = _lrelu(elv + erv)
                    M_ = _lrelu(mxvec + erv)
                    wv = jnp.exp(e_ - M_)
                    for hh in range(H):
                        bv = jnp.broadcast_to(wv[8 + hh], (16,))
                        for t in range(2):
                            col = hh * 32 + t * 16
                            wbuf[j, pl.ds(col, 16)] = bv * wbuf[j, pl.ds(col, 16)]
                    cur = wbuf[j, pl.ds(120, 16)]
                    wbuf[j, pl.ds(120, 16)] = jnp.where(lo8, cur, wv)

                for jj in range(B3 // 16):
                    d = dstv[pl.ds(jj * 16, 16)]
                    ok = (d >= base) & (d < hi)
                    idxv[pl.ds(jj * 16, 16)] = jnp.where(ok, d - base, SC3_CHUNK)
                pltpu.sync_copy(wbuf, accum.at[idxv], add=True)

                nxt = blk + 2

                @pl.when(nxt < NBLOCKS3_TILE)
                def _():
                    off2 = tile_off + nxt * B3
                    pltpu.sync_copy(src_hbm.at[pl.ds(off2, B3)], srcv)
                    pltpu.sync_copy(dst_hbm.at[pl.ds(off2, B3)], dstv)
                    pltpu.async_copy(whel_hbm.at[srcv], wbuf, semA)
                    pltpu.async_copy(dtab_hbm.at[dstv], dbuf, semB)

        plsc.subcore_barrier()

        @pl.when(s < 15)
        def _():
            pltpu.sync_copy(
                accum.at[pl.ds(s * SC3_STRIPE, SC3_STRIPE), :],
                out_hbm.at[pl.ds(base + s * SC3_STRIPE, SC3_STRIPE), :])

        full15 = 15 * SC3_STRIPE
        if k == 0:
            @pl.when(s == 15)
            def _():
                pltpu.sync_copy(
                    accum.at[pl.ds(full15, SC3_CHUNK - full15), :],
                    out_hbm.at[pl.ds(base + full15, SC3_CHUNK - full15), :])
        else:
            @pl.when((s == 15) & (c == 0))
            def _():
                pltpu.sync_copy(
                    accum.at[pl.ds(full15, SC3_CHUNK - full15), :],
                    out_hbm.at[pl.ds(base + full15, SC3_CHUNK - full15), :])

            @pl.when((s == 15) & (c == 1))
            def _():
                pltpu.sync_copy(
                    accum.at[pl.ds(full15, N - 3 * SC3_CHUNK - full15), :],
                    out_hbm.at[pl.ds(base + full15,
                                     N - 3 * SC3_CHUNK - full15), :])

        plsc.subcore_barrier()


def _sc3_call(whel, dtab, srcp, dstp, mx16, z3):
    return pl.kernel(
        _sc3_body,
        out_type=jax.ShapeDtypeStruct((N, SC3_AW), jnp.float32),
        mesh=_SC_MESH,
        compiler_params=_SC_PARAMS,
        scratch_types=[
            pltpu.VMEM((B3,), jnp.int32),
            pltpu.VMEM((B3,), jnp.int32),
            pltpu.VMEM((B3,), jnp.int32),
            pltpu.VMEM((B3,), jnp.int32),
            pltpu.VMEM((B3,), jnp.int32),
            pltpu.VMEM((B3,), jnp.int32),
            pltpu.VMEM((16,), jnp.float32),
            pltpu.VMEM((B3, 16), jnp.float32),
            pltpu.VMEM((B3, 16), jnp.float32),
            pltpu.VMEM((B3, SC3_W), jnp.float32),
            pltpu.VMEM((B3, SC3_W), jnp.float32),
            pltpu.SemaphoreType.DMA,
            pltpu.SemaphoreType.DMA,
            pltpu.SemaphoreType.DMA,
            pltpu.SemaphoreType.DMA,
            pltpu.VMEM_SHARED((SC3_ROWS, SC3_AW), jnp.float32),
        ],
    )(whel, dtab, srcp, dstp, mx16, z3)


def kernel(x, edge_index, edge_attr, Wf, bf, gf, betaf, We, be, ge, betae,
           Wfc, al, ar, bgat, g1, beta1, Wgate, bgate, Wo, bo, go, betao):
    f32 = jnp.float32
    xp = jnp.zeros((NP, 8), f32).at[:N, :7].set(x)
    Wf8 = jnp.zeros((8, H0), f32).at[:7].set(Wf)
    eap = jnp.zeros((EP, 8), f32).at[:E, :6].set(edge_attr)
    We8 = jnp.zeros((8, H0), f32).at[:6].set(We)
    srcp = jnp.concatenate([edge_index[0], jnp.zeros((EP - E,), jnp.int32)])
    dstp = jnp.concatenate([edge_index[1],
                            jnp.full((EP - E,), N, jnp.int32)])
    Al = jnp.zeros((H1, H), f32)
    Ar = jnp.zeros((H1, H), f32)
    for hh in range(H):
        Al = Al.at[hh * DH:(hh + 1) * DH, hh].set(al[hh])
        Ar = Ar.at[hh * DH:(hh + 1) * DH, hh].set(ar[hh])
    S4 = jnp.kron(jnp.eye(H, dtype=f32), jnp.ones((1, DH), f32))
    Wg8 = jnp.zeros((H1, 8), f32).at[:, 0:1].set(Wgate)
    bg8 = jnp.zeros((1, 8), f32).at[0, 0].set(bgate[0])
    r = lambda v: v.reshape(1, -1)
    z1 = jnp.zeros((SC1_STRIPE, SC1_W), f32)
    z3 = jnp.zeros((SC3_STRIPE, SC3_AW), f32)

    h = pl.pallas_call(
        _tc_face_body,
        grid=(NBLK,),
        in_specs=[
            pl.BlockSpec((RB, 8), lambda i: (i, 0)),
            pl.BlockSpec((8, H0), lambda i: (0, 0)),
            _row_spec(H0), _row_spec(H0), _row_spec(H0),
        ],
        out_specs=pl.BlockSpec((RB, H0), lambda i: (i, 0)),
        out_shape=jax.ShapeDtypeStruct((NP, H0), f32),
    )(xp, Wf8, r(bf), r(gf), r(betaf))

    eep = pl.pallas_call(
        _tc_edge_body,
        grid=(EBLK,),
        in_specs=[
            pl.BlockSpec((EB, 8), lambda i: (i, 0)),
            pl.BlockSpec((8, H0), lambda i: (0, 0)),
            _row_spec(H0), _row_spec(H0), _row_spec(H0),
        ],
        out_specs=pl.BlockSpec((EB, SC1_W), lambda i: (i, 0)),
        out_shape=jax.ShapeDtypeStruct((EP, SC1_W), f32),
    )(eap, We8, r(be), r(ge), r(betae))

    ssc = _sc1_call(eep, dstp, z1)
    sscp = jnp.zeros((NP, SC1_W), f32).at[:N].set(ssc)

    whel, dtab, mx16 = pl.pallas_call(
        _tc_mid_body,
        grid=(NBLK,),
        in_specs=[
            pl.BlockSpec((RB, H0), lambda i: (i, 0)),
            pl.BlockSpec((RB, SC1_W), lambda i: (i, 0)),
            pl.BlockSpec((H0, H1), lambda i: (0, 0)),
            pl.BlockSpec((H1, H), lambda i: (0, 0)),
            pl.BlockSpec((H1, H), lambda i: (0, 0)),
            _row_spec(H0), _row_spec(H0), _row_spec(H0),
        ],
        out_specs=[
            pl.BlockSpec((RB, SC3_W), lambda i: (i, 0)),
            pl.BlockSpec((RB, 16), lambda i: (i, 0)),
            pl.BlockSpec((1, 16), lambda i: (0, 0)),
        ],
        out_shape=[
            jax.ShapeDtypeStruct((NP, SC3_W), f32),
            jax.ShapeDtypeStruct((NP, 16), f32),
            jax.ShapeDtypeStruct((1, 16), f32),
        ],
    )(h, sscp, Wfc, Al, Ar, r(be), r(ge), r(betae))

    nd = _sc3_call(whel, dtab, srcp, dstp, mx16.reshape(16), z3)
    ndp = jnp.zeros((NP, SC3_AW), f32).at[:N].set(nd)

    nf, z8 = pl.pallas_call(
        _tc_post_body,
        grid=(NBLK,),
        in_specs=[
            pl.BlockSpec((RB, SC3_AW), lambda i: (i, 0)),
            pl.BlockSpec((RB, SC3_W), lambda i: (i, 0)),
            pl.BlockSpec((RB, 16), lambda i: (i, 0)),
            pl.BlockSpec((1, 16), lambda i: (0, 0)),
            pl.BlockSpec((H, H1), lambda i: (0, 0)),
            _row_spec(H1), _row_spec(H1), _row_spec(H1),
            pl.BlockSpec((H1, 8), lambda i: (0, 0)),
            _row_spec(8),
        ],
        out_specs=[
            pl.BlockSpec((RB, H1), lambda i: (i, 0)),
            pl.BlockSpec((RB, 8), lambda i: (i, 0)),
        ],
        out_shape=[
            jax.ShapeDtypeStruct((NP, H1), f32),
            jax.ShapeDtypeStruct((NP, 8), f32),
        ],
    )(ndp, whel, dtab, mx16, S4, r(bgat), r(g1), r(beta1), Wg8, bg8)

    zmax8 = pl.pallas_call(
        _tc_zmax_body,
        grid=(NBLK,),
        in_specs=[pl.BlockSpec((RB, 8), lambda i: (i, 0))],
        out_specs=pl.BlockSpec((1, 8), lambda i: (0, 0)),
        out_shape=jax.ShapeDtypeStruct((1, 8), f32),
    )(z8)

    pooled, feat = pl.pallas_call(
        _tc_pool_body,
        grid=(NBLK,),
        in_specs=[
            pl.BlockSpec((RB, H1), lambda i: (i, 0)),
            pl.BlockSpec((RB, 8), lambda i: (i, 0)),
            pl.BlockSpec((1, 8), lambda i: (0, 0)),
            pl.BlockSpec((H1, H1), lambda i: (0, 0)),
            _row_spec(H1), _row_spec(H1), _row_spec(H1),
        ],
        out_specs=[
            pl.BlockSpec((1, H1), lambda i: (0, 0)),
            pl.BlockSpec((1, H1), lambda i: (0, 0)),
        ],
        out_shape=[
            jax.ShapeDtypeStruct((1, H1), f32),
            jax.ShapeDtypeStruct((1, H1), f32),
        ],
        scratch_shapes=[
            pltpu.VMEM((1, H1), f32),
            pltpu.VMEM((1, 8), f32),
        ],
    )(nf, z8, zmax8, Wo, r(bo), r(go), r(betao))

    return (feat, nf[:N], pooled)


# speedup vs baseline: 1.5367x; 1.0004x over previous
import functools

import jax
import jax.numpy as jnp
from jax import lax
from jax.experimental import pallas as pl
from jax.experimental.pallas import tpu as pltpu
from jax.experimental.pallas import tpu_sc as plsc

N = 50000
E = 800000
H = 4
DH = 32
H0 = 64
H1 = 128

NP = 50176
EP = 819200
NBLK = 98
RB = 512
EB = 1024
EBLK = EP // EB

SC1_HALF = 25000
SC1_ROWS = 25088
SC1_STRIPE = 1568
SC1_W = 72
SC3_CHUNK = 12512
SC3_ROWS = 12544
SC3_STRIPE = 784
SC3_W = 136
SC3_AW = 136
EDGES_PER_TILE = EP // 16
B = 128
NBLOCKS_TILE = EDGES_PER_TILE // B
B3 = 64
NBLOCKS3_TILE = EDGES_PER_TILE // B3


def _ln(h, g, b, eps=1e-5):
    mu = jnp.mean(h, axis=-1, keepdims=True)
    va = jnp.mean((h - mu) * (h - mu), axis=-1, keepdims=True)
    return (h - mu) / jnp.sqrt(va + eps) * g + b


def _lrelu(v):
    return jnp.maximum(v, 0.2 * v)


def _tc_face_body(xb, Wf8, bf, gf, betaf, hb):
    y = jax.nn.relu(jnp.dot(xb[...], Wf8[...],
                            preferred_element_type=jnp.float32) + bf[...])
    hb[...] = _ln(y, gf[...], betaf[...])


def _tc_edge_body(eab, We8, be, ge, betae, eepb):
    y = jax.nn.relu(jnp.dot(eab[...], We8[...],
                            preferred_element_type=jnp.float32) + be[...])
    ee = _ln(y, ge[...], betae[...])
    ones = jnp.ones((EB, 1), jnp.float32)
    zeros = jnp.zeros((EB, SC1_W - H0 - 1), jnp.float32)
    eepb[...] = jnp.concatenate([ee, ones, zeros], axis=1)


def _tc_mid_body(hb, sscb, Wfc, Al, Ar, be, ge, betae, whelb, dtabb, mx16b):
    i = pl.program_id(0)
    c0 = _ln(jax.nn.relu(be[...]), ge[...], betae[...])
    ssum = sscb[:, :H0]
    cnt = sscb[:, H0:H0 + 1]
    h2 = hb[...] + (ssum + c0) / (cnt + 1.0)
    Wh = jnp.dot(h2, Wfc[...], preferred_element_type=jnp.float32)
    el = jnp.dot(Wh, Al[...], preferred_element_type=jnp.float32)
    er = jnp.dot(Wh, Ar[...], preferred_element_type=jnp.float32)
    z8 = jnp.zeros((RB, 8), jnp.float32)
    z4 = jnp.zeros((RB, 4), jnp.float32)
    whelb[...] = jnp.concatenate([Wh, el, z4], axis=1)
    dtabb[...] = jnp.concatenate([z8, er, z4], axis=1)
    elz = jnp.concatenate([z8, el, z4], axis=1)
    bm = jnp.max(elz, axis=0, keepdims=True)

    @pl.when(i == 0)
    def _():
        mx16b[...] = jnp.full((1, 16), -1e30, jnp.float32)

    mx16b[...] = jnp.maximum(mx16b[...], bm)


def _tc_post_body(ndb, whelb, dtabb, mx16b, S4, bgat, g1, beta1, Wg8, bg8,
                  nfb, z8b):
    Wh = whelb[:, :H1]
    elv = whelb[:, H1:H1 + H]
    erv = dtabb[:, 8:12]
    mx = mx16b[:, 8:12]
    sE = _lrelu(elv + erv)
    M = _lrelu(mx + erv)
    sw = jnp.exp(sE - M)
    num = ndb[:, :H1] + jnp.dot(sw, S4[...],
                                preferred_element_type=jnp.float32) * Wh
    den = ndb[:, H1:H1 + H] + sw
    out = num / (jnp.dot(den, S4[...],
                         preferred_element_type=jnp.float32) + 1e-9)
    out = jax.nn.relu(out + bgat[...])
    nf = _ln(out, g1[...], beta1[...])
    nfb[...] = nf
    z8b[...] = jnp.dot(nf, Wg8[...], preferred_element_type=jnp.float32) + bg8[...]


def _tc_zmax_body(z8b, zmaxb):
    i = pl.program_id(0)
    rows = lax.broadcasted_iota(jnp.int32, (RB, 1), 0) + i * RB
    zm = jnp.where(rows < N, z8b[:, 0:1], -1e30)
    bm = jnp.max(zm)

    @pl.when(i == 0)
    def _():
        zmaxb[...] = jnp.full((1, 8), -1e30, jnp.float32)

    zmaxb[...] = jnp.maximum(zmaxb[...], bm)


def _tc_pool_body(nfb, z8b, zmaxb, Wo, bo, go, betao, pooledb, featb,
                  Sv, S1):
    i = pl.program_id(0)

    @pl.when(i == 0)
    def _():
        Sv[...] = jnp.zeros((1, H1), jnp.float32)
        S1[...] = jnp.zeros((1, 8), jnp.float32)

    rows = lax.broadcasted_iota(jnp.int32, (RB, 1), 0) + i * RB
    ge_ = jnp.where(rows < N, jnp.exp(z8b[:, 0:1] - zmaxb[:, 0:1]), 0.0)
    Sv[...] += jnp.sum(ge_ * nfb[...], axis=0, keepdims=True)
    S1[...] += jnp.full((1, 8), jnp.sum(ge_), jnp.float32)

    @pl.when(i == NBLK - 1)
    def _():
        pooled = Sv[...] / S1[:, 0:1]
        feat = jax.nn.relu(jnp.dot(pooled, Wo[...],
                                   preferred_element_type=jnp.float32) + bo[...])
        featb[...] = _ln(feat, go[...], betao[...])
        pooledb[...] = pooled


def _row_spec(w):
    return pl.BlockSpec((1, w), lambda i: (0, 0))


_SC_MESH = plsc.VectorSubcoreMesh(core_axis_name="c", subcore_axis_name="s")
_SC_PARAMS = pltpu.CompilerParams(use_tc_tiling_on_sc=False)


def _sc1_body(eep_hbm, dst_hbm, z1_hbm, out_hbm, dstv, idxv, valv, accum):
    c = lax.axis_index("c")
    s = lax.axis_index("s")
    base = c * SC1_HALF
    pltpu.sync_copy(z1_hbm, accum.at[pl.ds(s * SC1_STRIPE, SC1_STRIPE), :])
    plsc.subcore_barrier()

    tile_off = s * EDGES_PER_TILE

    @pl.loop(0, NBLOCKS_TILE)
    def _blk(blk):
        off = tile_off + blk * B
        pltpu.sync_copy(dst_hbm.at[pl.ds(off, B)], dstv)
        pltpu.sync_copy(eep_hbm.at[pl.ds(off, B), :], valv)
        for j in range(B // 16):
            d = dstv[pl.ds(j * 16, 16)]
            ok = (d >= base) & (d < base + SC1_HALF)
            idxv[pl.ds(j * 16, 16)] = jnp.where(ok, d - base, SC1_HALF)
        pltpu.sync_copy(valv, accum.at[idxv], add=True)

    plsc.subcore_barrier()
    @pl.when(s < 15)
    def _():
        pltpu.sync_copy(
            accum.at[pl.ds(s * SC1_STRIPE, SC1_STRIPE), :],
            out_hbm.at[pl.ds(base + s * SC1_STRIPE, SC1_STRIPE), :])

    @pl.when(s == 15)
    def _():
        pltpu.sync_copy(
            accum.at[pl.ds(15 * SC1_STRIPE, SC1_HALF - 15 * SC1_STRIPE), :],
            out_hbm.at[pl.ds(base + 15 * SC1_STRIPE,
                             SC1_HALF - 15 * SC1_STRIPE), :])


def _sc1_call(eep, dstp, z1):
    return pl.kernel(
        _sc1_body,
        out_type=jax.ShapeDtypeStruct((N, SC1_W), jnp.float32),
        mesh=_SC_MESH,
        compiler_params=_SC_PARAMS,
        scratch_types=[
            pltpu.VMEM((B,), jnp.int32),
            pltpu.VMEM((B,), jnp.int32),
            pltpu.VMEM((B, SC1_W), jnp.float32),
            pltpu.VMEM_SHARED((SC1_ROWS, SC1_W), jnp.float32),
        ],
    )(eep, dstp, z1)


def _sc3_body(whel_hbm, dtab_hbm, src_hbm, dst_hbm, mx_hbm, z3_hbm, out_hbm,
              srcv0, srcv1, dstv0, dstv1, idxv0, idxv1, mxv,
              dbuf0, dbuf1, wbuf0, wbuf1,
              semA0, semA1, semB0, semB1, accum):
    c = lax.axis_index("c")
    s = lax.axis_index("s")
    tile_off = s * EDGES_PER_TILE
    pltpu.sync_copy(mx_hbm, mxv)
    mxvec = mxv[...]
    bufs = ((srcv0, dstv0, idxv0, dbuf0, wbuf0, semA0, semB0),
            (srcv1, dstv1, idxv1, dbuf1, wbuf1, semA1, semB1))

    for k in range(2):
        chunk = c * 2 + k
        base = chunk * SC3_CHUNK
        hi = jnp.minimum(base + SC3_CHUNK, N)
        pltpu.sync_copy(z3_hbm, accum.at[pl.ds(s * SC3_STRIPE, SC3_STRIPE), :])
        plsc.subcore_barrier()

        for b in range(2):
            srcv, dstv, idxv, dbuf, wbuf, semA, semB = bufs[b]
            off = tile_off + b * B3
            pltpu.sync_copy(src_hbm.at[pl.ds(off, B3)], srcv)
            pltpu.sync_copy(dst_hbm.at[pl.ds(off, B3)], dstv)
            pltpu.async_copy(whel_hbm.at[srcv], wbuf, semA)
            pltpu.async_copy(dtab_hbm.at[dstv], dbuf, semB)

        @pl.loop(0, NBLOCKS3_TILE, step=2)
        def _blk(g):
            for b in range(2):
                srcv, dstv, idxv, dbuf, wbuf, semA, semB = bufs[b]
                blk = g + b
                pltpu.make_async_copy(whel_hbm.at[srcv], wbuf, semA).wait()
                pltpu.make_async_copy(dtab_hbm.at[dstv], dbuf, semB).wait()

                lane = lax.iota(jnp.int32, 16)
                lo8 = lane < 8

                @pl.loop(0, B3, unroll=8)
                def _edge(j):
                    elv = wbuf[j, pl.ds(120, 16)]
                    erv = dbuf[j, pl.ds(0, 16)]
                    e_ = _lrelu(elv + erv)
                    M_ = _lrelu(mxvec + erv)
                    wv = jnp.exp(e_ - M_)
                    for hh in range(H):
                        bv = jnp.broadcast_to(wv[8 + hh], (16,))
                        for t in range(2):
                            col = hh * 32 + t * 16
                            wbuf[j, pl.ds(col, 16)] = bv * wbuf[j, pl.ds(col, 16)]
                    cur = wbuf[j, pl.ds(120, 16)]
                    wbuf[j, pl.ds(120, 16)] = jnp.where(lo8, cur, wv)

                for jj in range(B3 // 16):
                    d = dstv[pl.ds(jj * 16, 16)]
                    ok = (d >= base) & (d < hi)
                    idxv[pl.ds(jj * 16, 16)] = jnp.where(ok, d - base, SC3_CHUNK)
                pltpu.sync_copy(wbuf, accum.at[idxv], add=True)

                nxt = blk + 2

                @pl.when(nxt < NBLOCKS3_TILE)
                def _():
                    off2 = tile_off + nxt * B3
                    pltpu.sync_copy(src_hbm.at[pl.ds(off2, B3)], srcv)
                    pltpu.sync_copy(dst_hbm.at[pl.ds(off2, B3)], dstv)
                    pltpu.async_copy(whel_hbm.at[srcv], wbuf, semA)
                    pltpu.async_copy(dtab_hbm.at[dstv], dbuf, semB)

        plsc.subcore_barrier()

        @pl.when(s < 15)
        def _():
            pltpu.sync_copy(
                accum.at[pl.ds(s * SC3_STRIPE, SC3_STRIPE), :],
                out_hbm.at[pl.ds(base + s * SC3_STRIPE, SC3_STRIPE), :])

        full15 = 15 * SC3_STRIPE
        if k == 0:
            @pl.when(s == 15)
            def _():
                pltpu.sync_copy(
                    accum.at[pl.ds(full15, SC3_CHUNK - full15), :],
                    out_hbm.at[pl.ds(base + full15, SC3_CHUNK - full15), :])
        else:
            @pl.when((s == 15) & (c == 0))
            def _():
                pltpu.sync_copy(
                    accum.at[pl.ds(full15, SC3_CHUNK - full15), :],
                    out_hbm.at[pl.ds(base + full15, SC3_CHUNK - full15), :])

            @pl.when((s == 15) & (c == 1))
            def _():
                pltpu.sync_copy(
                    accum.at[pl.ds(full15, N - 3 * SC3_CHUNK - full15), :],
                    out_hbm.at[pl.ds(base + full15,
                                     N - 3 * SC3_CHUNK - full15), :])

        plsc.subcore_barrier()


def _sc3_call(whel, dtab, srcp, dstp, mx16, z3):
    return pl.kernel(
        _sc3_body,
        out_type=jax.ShapeDtypeStruct((N, SC3_AW), jnp.float32),
        mesh=_SC_MESH,
        compiler_params=_SC_PARAMS,
        scratch_types=[
            pltpu.VMEM((B3,), jnp.int32),
            pltpu.VMEM((B3,), jnp.int32),
            pltpu.VMEM((B3,), jnp.int32),
            pltpu.VMEM((B3,), jnp.int32),
            pltpu.VMEM((B3,), jnp.int32),
            pltpu.VMEM((B3,), jnp.int32),
            pltpu.VMEM((16,), jnp.float32),
            pltpu.VMEM((B3, 16), jnp.float32),
            pltpu.VMEM((B3, 16), jnp.float32),
            pltpu.VMEM((B3, SC3_W), jnp.float32),
            pltpu.VMEM((B3, SC3_W), jnp.float32),
            pltpu.SemaphoreType.DMA,
            pltpu.SemaphoreType.DMA,
            pltpu.SemaphoreType.DMA,
            pltpu.SemaphoreType.DMA,
            pltpu.VMEM_SHARED((SC3_ROWS, SC3_AW), jnp.float32),
        ],
    )(whel, dtab, srcp, dstp, mx16, z3)


def kernel(x, edge_index, edge_attr, Wf, bf, gf, betaf, We, be, ge, betae,
           Wfc, al, ar, bgat, g1, beta1, Wgate, bgate, Wo, bo, go, betao):
    f32 = jnp.float32
    xp = jnp.zeros((NP, 8), f32).at[:N, :7].set(x)
    Wf8 = jnp.zeros((8, H0), f32).at[:7].set(Wf)
    eap = jnp.zeros((EP, 8), f32).at[:E, :6].set(edge_attr)
    We8 = jnp.zeros((8, H0), f32).at[:6].set(We)
    srcp = jnp.concatenate([edge_index[0], jnp.zeros((EP - E,), jnp.int32)])
    dstp = jnp.concatenate([edge_index[1],
                            jnp.full((EP - E,), N, jnp.int32)])
    Al = jnp.zeros((H1, H), f32)
    Ar = jnp.zeros((H1, H), f32)
    for hh in range(H):
        Al = Al.at[hh * DH:(hh + 1) * DH, hh].set(al[hh])
        Ar = Ar.at[hh * DH:(hh + 1) * DH, hh].set(ar[hh])
    S4 = jnp.kron(jnp.eye(H, dtype=f32), jnp.ones((1, DH), f32))
    Wg8 = jnp.zeros((H1, 8), f32).at[:, 0:1].set(Wgate)
    bg8 = jnp.zeros((1, 8), f32).at[0, 0].set(bgate[0])
    r = lambda v: v.reshape(1, -1)
    z1 = jnp.zeros((SC1_STRIPE, SC1_W), f32)
    z3 = jnp.zeros((SC3_STRIPE, SC3_AW), f32)

    h = pl.pallas_call(
        _tc_face_body,
        grid=(NBLK,),
        in_specs=[
            pl.BlockSpec((RB, 8), lambda i: (i, 0)),
            pl.BlockSpec((8, H0), lambda i: (0, 0)),
            _row_spec(H0), _row_spec(H0), _row_spec(H0),
        ],
        out_specs=pl.BlockSpec((RB, H0), lambda i: (i, 0)),
        out_shape=jax.ShapeDtypeStruct((NP, H0), f32),
    )(xp, Wf8, r(bf), r(gf), r(betaf))

    eep = pl.pallas_call(
        _tc_edge_body,
        grid=(EBLK,),
        in_specs=[
            pl.BlockSpec((EB, 8), lambda i: (i, 0)),
            pl.BlockSpec((8, H0), lambda i: (0, 0)),
            _row_spec(H0), _row_spec(H0), _row_spec(H0),
        ],
        out_specs=pl.BlockSpec((EB, SC1_W), lambda i: (i, 0)),
        out_shape=jax.ShapeDtypeStruct((EP, SC1_W), f32),
    )(eap, We8, r(be), r(ge), r(betae))

    ssc = _sc1_call(eep, dstp, z1)
    sscp = jnp.zeros((NP, SC1_W), f32).at[:N].set(ssc)

    whel, dtab, mx16 = pl.pallas_call(
        _tc_mid_body,
        grid=(NBLK,),
        in_specs=[
            pl.BlockSpec((RB, H0), lambda i: (i, 0)),
            pl.BlockSpec((RB, SC1_W), lambda i: (i, 0)),
            pl.BlockSpec((H0, H1), lambda i: (0, 0)),
            pl.BlockSpec((H1, H), lambda i: (0, 0)),
            pl.BlockSpec((H1, H), lambda i: (0, 0)),
            _row_spec(H0), _row_spec(H0), _row_spec(H0),
        ],
        out_specs=[
            pl.BlockSpec((RB, SC3_W), lambda i: (i, 0)),
            pl.BlockSpec((RB, 16), lambda i: (i, 0)),
            pl.BlockSpec((1, 16), lambda i: (0, 0)),
        ],
        out_shape=[
            jax.ShapeDtypeStruct((NP, SC3_W), f32),
            jax.ShapeDtypeStruct((NP, 16), f32),
            jax.ShapeDtypeStruct((1, 16), f32),
        ],
    )(h, sscp, Wfc, Al, Ar, r(be), r(ge), r(betae))

    nd = _sc3_call(whel, dtab, srcp, dstp, mx16.reshape(16), z3)
    ndp = jnp.zeros((NP, SC3_AW), f32).at[:N].set(nd)

    nf, z8 = pl.pallas_call(
        _tc_post_body,
        grid=(NBLK,),
        in_specs=[
            pl.BlockSpec((RB, SC3_AW), lambda i: (i, 0)),
            pl.BlockSpec((RB, SC3_W), lambda i: (i, 0)),
            pl.BlockSpec((RB, 16), lambda i: (i, 0)),
            pl.BlockSpec((1, 16), lambda i: (0, 0)),
            pl.BlockSpec((H, H1), lambda i: (0, 0)),
            _row_spec(H1), _row_spec(H1), _row_spec(H1),
            pl.BlockSpec((H1, 8), lambda i: (0, 0)),
            _row_spec(8),
        ],
        out_specs=[
            pl.BlockSpec((RB, H1), lambda i: (i, 0)),
            pl.BlockSpec((RB, 8), lambda i: (i, 0)),
        ],
        out_shape=[
            jax.ShapeDtypeStruct((NP, H1), f32),
            jax.ShapeDtypeStruct((NP, 8), f32),
        ],
    )(ndp, whel, dtab, mx16, S4, r(bgat), r(g1), r(beta1), Wg8, bg8)

    zmax8 = pl.pallas_call(
        _tc_zmax_body,
        grid=(NBLK,),
        in_specs=[pl.BlockSpec((RB, 8), lambda i: (i, 0))],
        out_specs=pl.BlockSpec((1, 8), lambda i: (0, 0)),
        out_shape=jax.ShapeDtypeStruct((1, 8), f32),
    )(z8)

    pooled, feat = pl.pallas_call(
        _tc_pool_body,
        grid=(NBLK,),
        in_specs=[
            pl.BlockSpec((RB, H1), lambda i: (i, 0)),
            pl.BlockSpec((RB, 8), lambda i: (i, 0)),
            pl.BlockSpec((1, 8), lambda i: (0, 0)),
            pl.BlockSpec((H1, H1), lambda i: (0, 0)),
            _row_spec(H1), _row_spec(H1), _row_spec(H1),
        ],
        out_specs=[
            pl.BlockSpec((1, H1), lambda i: (0, 0)),
            pl.BlockSpec((1, H1), lambda i: (0, 0)),
        ],
        out_shape=[
            jax.ShapeDtypeStruct((1, H1), f32),
            jax.ShapeDtypeStruct((1, H1), f32),
        ],
        scratch_shapes=[
            pltpu.VMEM((1, H1), f32),
            pltpu.VMEM((1, 8), f32),
        ],
    )(nf, z8, zmax8, Wo, r(bo), r(go), r(betao))

    return (feat, nf[:N], pooled)


# speedup vs baseline: 1.5613x; 1.0160x over previous
import functools

import jax
import jax.numpy as jnp
from jax import lax
from jax.experimental import pallas as pl
from jax.experimental.pallas import tpu as pltpu
from jax.experimental.pallas import tpu_sc as plsc

N = 50000
E = 800000
H = 4
DH = 32
H0 = 64
H1 = 128

NP = 50176
EP = 819200
NBLK = 98
RB = 512
EB = 1024
EBLK = EP // EB

SC1_HALF = 25000
SC1_ROWS = 25088
SC1_STRIPE = 1568
SC1_W = 72
SC3_CHUNK = 12512
SC3_ROWS = 12544
SC3_STRIPE = 784
SC3_W = 136
SC3_AW = 136
EDGES_PER_TILE = EP // 16
B = 64
NBLOCKS_TILE = EDGES_PER_TILE // B
B3 = 64
NBLOCKS3_TILE = EDGES_PER_TILE // B3


def _ln(h, g, b, eps=1e-5):
    mu = jnp.mean(h, axis=-1, keepdims=True)
    va = jnp.mean((h - mu) * (h - mu), axis=-1, keepdims=True)
    return (h - mu) / jnp.sqrt(va + eps) * g + b


def _lrelu(v):
    return jnp.maximum(v, 0.2 * v)


def _tc_face_body(xb, Wf8, bf, gf, betaf, hb):
    y = jax.nn.relu(jnp.dot(xb[...], Wf8[...],
                            preferred_element_type=jnp.float32) + bf[...])
    hb[...] = _ln(y, gf[...], betaf[...])


def _tc_edge_body(eab, We8, be, ge, betae, eepb):
    y = jax.nn.relu(jnp.dot(eab[...], We8[...],
                            preferred_element_type=jnp.float32) + be[...])
    ee = _ln(y, ge[...], betae[...])
    ones = jnp.ones((EB, 1), jnp.float32)
    zeros = jnp.zeros((EB, SC1_W - H0 - 1), jnp.float32)
    eepb[...] = jnp.concatenate([ee, ones, zeros], axis=1)


def _tc_mid_body(hb, sscb, Wfc, Al, Ar, be, ge, betae, whelb, dtabb, mx16b):
    i = pl.program_id(0)
    c0 = _ln(jax.nn.relu(be[...]), ge[...], betae[...])
    ssum = sscb[:, :H0]
    cnt = sscb[:, H0:H0 + 1]
    h2 = hb[...] + (ssum + c0) / (cnt + 1.0)
    Wh = jnp.dot(h2, Wfc[...], preferred_element_type=jnp.float32)
    el = jnp.dot(Wh, Al[...], preferred_element_type=jnp.float32)
    er = jnp.dot(Wh, Ar[...], preferred_element_type=jnp.float32)
    z8 = jnp.zeros((RB, 8), jnp.float32)
    z4 = jnp.zeros((RB, 4), jnp.float32)
    whelb[...] = jnp.concatenate([Wh, el, z4], axis=1)
    dtabb[...] = jnp.concatenate([z8, er, z4], axis=1)
    elz = jnp.concatenate([z8, el, z4], axis=1)
    bm = jnp.max(elz, axis=0, keepdims=True)

    @pl.when(i == 0)
    def _():
        mx16b[...] = jnp.full((1, 16), -1e30, jnp.float32)

    mx16b[...] = jnp.maximum(mx16b[...], bm)


def _tc_post_body(ndb, whelb, dtabb, mx16b, S4, bgat, g1, beta1, Wg8, bg8,
                  nfb, z8b):
    Wh = whelb[:, :H1]
    elv = whelb[:, H1:H1 + H]
    erv = dtabb[:, 8:12]
    mx = mx16b[:, 8:12]
    sE = _lrelu(elv + erv)
    M = _lrelu(mx + erv)
    sw = jnp.exp(sE - M)
    num = ndb[:, :H1] + jnp.dot(sw, S4[...],
                                preferred_element_type=jnp.float32) * Wh
    den = ndb[:, H1:H1 + H] + sw
    out = num / (jnp.dot(den, S4[...],
                         preferred_element_type=jnp.float32) + 1e-9)
    out = jax.nn.relu(out + bgat[...])
    nf = _ln(out, g1[...], beta1[...])
    nfb[...] = nf
    z8b[...] = jnp.dot(nf, Wg8[...], preferred_element_type=jnp.float32) + bg8[...]


def _tc_zmax_body(z8b, zmaxb):
    i = pl.program_id(0)
    rows = lax.broadcasted_iota(jnp.int32, (RB, 1), 0) + i * RB
    zm = jnp.where(rows < N, z8b[:, 0:1], -1e30)
    bm = jnp.max(zm)

    @pl.when(i == 0)
    def _():
        zmaxb[...] = jnp.full((1, 8), -1e30, jnp.float32)

    zmaxb[...] = jnp.maximum(zmaxb[...], bm)


def _tc_pool_body(nfb, z8b, zmaxb, Wo, bo, go, betao, pooledb, featb,
                  Sv, S1):
    i = pl.program_id(0)

    @pl.when(i == 0)
    def _():
        Sv[...] = jnp.zeros((1, H1), jnp.float32)
        S1[...] = jnp.zeros((1, 8), jnp.float32)

    rows = lax.broadcasted_iota(jnp.int32, (RB, 1), 0) + i * RB
    ge_ = jnp.where(rows < N, jnp.exp(z8b[:, 0:1] - zmaxb[:, 0:1]), 0.0)
    Sv[...] += jnp.sum(ge_ * nfb[...], axis=0, keepdims=True)
    S1[...] += jnp.full((1, 8), jnp.sum(ge_), jnp.float32)

    @pl.when(i == NBLK - 1)
    def _():
        pooled = Sv[...] / S1[:, 0:1]
        feat = jax.nn.relu(jnp.dot(pooled, Wo[...],
                                   preferred_element_type=jnp.float32) + bo[...])
        featb[...] = _ln(feat, go[...], betao[...])
        pooledb[...] = pooled


def _row_spec(w):
    return pl.BlockSpec((1, w), lambda i: (0, 0))


_SC_MESH = plsc.VectorSubcoreMesh(core_axis_name="c", subcore_axis_name="s")
_SC_PARAMS = pltpu.CompilerParams(use_tc_tiling_on_sc=False)


def _sc1_body(eep_hbm, dst_hbm, z1_hbm, out_hbm,
              dstv0, dstv1, idxv0, idxv1, valv0, valv1, semV0, semV1, accum):
    c = lax.axis_index("c")
    s = lax.axis_index("s")
    base = c * SC1_HALF
    pltpu.sync_copy(z1_hbm, accum.at[pl.ds(s * SC1_STRIPE, SC1_STRIPE), :])
    plsc.subcore_barrier()

    tile_off = s * EDGES_PER_TILE
    bufs = ((dstv0, idxv0, valv0, semV0), (dstv1, idxv1, valv1, semV1))

    for b in range(2):
        dstv, idxv, valv, semV = bufs[b]
        off = tile_off + b * B
        pltpu.sync_copy(dst_hbm.at[pl.ds(off, B)], dstv)
        pltpu.async_copy(eep_hbm.at[pl.ds(off, B), :], valv, semV)

    @pl.loop(0, NBLOCKS_TILE, step=2)
    def _blk(g):
        for b in range(2):
            dstv, idxv, valv, semV = bufs[b]
            blk = g + b
            off = tile_off + blk * B
            pltpu.make_async_copy(
                eep_hbm.at[pl.ds(off, B), :], valv, semV).wait()
            for j in range(B // 16):
                d = dstv[pl.ds(j * 16, 16)]
                ok = (d >= base) & (d < base + SC1_HALF)
                idxv[pl.ds(j * 16, 16)] = jnp.where(ok, d - base, SC1_HALF)
            pltpu.sync_copy(valv, accum.at[idxv], add=True)
            nxt = blk + 2

            @pl.when(nxt < NBLOCKS_TILE)
            def _():
                off2 = tile_off + nxt * B
                pltpu.sync_copy(dst_hbm.at[pl.ds(off2, B)], dstv)
                pltpu.async_copy(eep_hbm.at[pl.ds(off2, B), :], valv, semV)

    plsc.subcore_barrier()
    @pl.when(s < 15)
    def _():
        pltpu.sync_copy(
            accum.at[pl.ds(s * SC1_STRIPE, SC1_STRIPE), :],
            out_hbm.at[pl.ds(base + s * SC1_STRIPE, SC1_STRIPE), :])

    @pl.when(s == 15)
    def _():
        pltpu.sync_copy(
            accum.at[pl.ds(15 * SC1_STRIPE, SC1_HALF - 15 * SC1_STRIPE), :],
            out_hbm.at[pl.ds(base + 15 * SC1_STRIPE,
                             SC1_HALF - 15 * SC1_STRIPE), :])


def _sc1_call(eep, dstp, z1):
    return pl.kernel(
        _sc1_body,
        out_type=jax.ShapeDtypeStruct((N, SC1_W), jnp.float32),
        mesh=_SC_MESH,
        compiler_params=_SC_PARAMS,
        scratch_types=[
            pltpu.VMEM((B,), jnp.int32),
            pltpu.VMEM((B,), jnp.int32),
            pltpu.VMEM((B,), jnp.int32),
            pltpu.VMEM((B,), jnp.int32),
            pltpu.VMEM((B, SC1_W), jnp.float32),
            pltpu.VMEM((B, SC1_W), jnp.float32),
            pltpu.SemaphoreType.DMA,
            pltpu.SemaphoreType.DMA,
            pltpu.VMEM_SHARED((SC1_ROWS, SC1_W), jnp.float32),
        ],
    )(eep, dstp, z1)


def _sc3_body(whel_hbm, dtab_hbm, src_hbm, dst_hbm, mx_hbm, z3_hbm, out_hbm,
              srcv0, srcv1, dstv0, dstv1, idxv0, idxv1, mxv,
              dbuf0, dbuf1, wbuf0, wbuf1,
              semA0, semA1, semB0, semB1, accum):
    c = lax.axis_index("c")
    s = lax.axis_index("s")
    tile_off = s * EDGES_PER_TILE
    pltpu.sync_copy(mx_hbm, mxv)
    mxvec = mxv[...]
    bufs = ((srcv0, dstv0, idxv0, dbuf0, wbuf0, semA0, semB0),
            (srcv1, dstv1, idxv1, dbuf1, wbuf1, semA1, semB1))

    for k in range(2):
        chunk = c * 2 + k
        base = chunk * SC3_CHUNK
        hi = jnp.minimum(base + SC3_CHUNK, N)
        pltpu.sync_copy(z3_hbm, accum.at[pl.ds(s * SC3_STRIPE, SC3_STRIPE), :])
        plsc.subcore_barrier()

        for b in range(2):
            srcv, dstv, idxv, dbuf, wbuf, semA, semB = bufs[b]
            off = tile_off + b * B3
            pltpu.sync_copy(src_hbm.at[pl.ds(off, B3)], srcv)
            pltpu.sync_copy(dst_hbm.at[pl.ds(off, B3)], dstv)
            pltpu.async_copy(whel_hbm.at[srcv], wbuf, semA)
            pltpu.async_copy(dtab_hbm.at[dstv], dbuf, semB)

        @pl.loop(0, NBLOCKS3_TILE, step=2)
        def _blk(g):
            for b in range(2):
                srcv, dstv, idxv, dbuf, wbuf, semA, semB = bufs[b]
                blk = g + b
                pltpu.make_async_copy(whel_hbm.at[srcv], wbuf, semA).wait()
                pltpu.make_async_copy(dtab_hbm.at[dstv], dbuf, semB).wait()

                lane = lax.iota(jnp.int32, 16)
                lo8 = lane < 8

                @pl.loop(0, B3, unroll=8)
                def _edge(j):
                    elv = wbuf[j, pl.ds(120, 16)]
                    erv = dbuf[j, pl.ds(0, 16)]
                    e_ = _lrelu(elv + erv)
                    M_ = _lrelu(mxvec + erv)
                    wv = jnp.exp(e_ - M_)
                    for hh in range(H):
                        bv = jnp.broadcast_to(wv[8 + hh], (16,))
                        for t in range(2):
                            col = hh * 32 + t * 16
                            wbuf[j, pl.ds(col, 16)] = bv * wbuf[j, pl.ds(col, 16)]
                    cur = wbuf[j, pl.ds(120, 16)]
                    wbuf[j, pl.ds(120, 16)] = jnp.where(lo8, cur, wv)

                for jj in range(B3 // 16):
                    d = dstv[pl.ds(jj * 16, 16)]
                    ok = (d >= base) & (d < hi)
                    idxv[pl.ds(jj * 16, 16)] = jnp.where(ok, d - base, SC3_CHUNK)
                pltpu.sync_copy(wbuf, accum.at[idxv], add=True)

                nxt = blk + 2

                @pl.when(nxt < NBLOCKS3_TILE)
                def _():
                    off2 = tile_off + nxt * B3
                    pltpu.sync_copy(src_hbm.at[pl.ds(off2, B3)], srcv)
                    pltpu.sync_copy(dst_hbm.at[pl.ds(off2, B3)], dstv)
                    pltpu.async_copy(whel_hbm.at[srcv], wbuf, semA)
                    pltpu.async_copy(dtab_hbm.at[dstv], dbuf, semB)

        plsc.subcore_barrier()

        @pl.when(s < 15)
        def _():
            pltpu.sync_copy(
                accum.at[pl.ds(s * SC3_STRIPE, SC3_STRIPE), :],
                out_hbm.at[pl.ds(base + s * SC3_STRIPE, SC3_STRIPE), :])

        full15 = 15 * SC3_STRIPE
        if k == 0:
            @pl.when(s == 15)
            def _():
                pltpu.sync_copy(
                    accum.at[pl.ds(full15, SC3_CHUNK - full15), :],
                    out_hbm.at[pl.ds(base + full15, SC3_CHUNK - full15), :])
        else:
            @pl.when((s == 15) & (c == 0))
            def _():
                pltpu.sync_copy(
                    accum.at[pl.ds(full15, SC3_CHUNK - full15), :],
                    out_hbm.at[pl.ds(base + full15, SC3_CHUNK - full15), :])

            @pl.when((s == 15) & (c == 1))
            def _():
                pltpu.sync_copy(
                    accum.at[pl.ds(full15, N - 3 * SC3_CHUNK - full15), :],
                    out_hbm.at[pl.ds(base + full15,
                                     N - 3 * SC3_CHUNK - full15), :])

        plsc.subcore_barrier()


def _sc3_call(whel, dtab, srcp, dstp, mx16, z3):
    return pl.kernel(
        _sc3_body,
        out_type=jax.ShapeDtypeStruct((N, SC3_AW), jnp.float32),
        mesh=_SC_MESH,
        compiler_params=_SC_PARAMS,
        scratch_types=[
            pltpu.VMEM((B3,), jnp.int32),
            pltpu.VMEM((B3,), jnp.int32),
            pltpu.VMEM((B3,), jnp.int32),
            pltpu.VMEM((B3,), jnp.int32),
            pltpu.VMEM((B3,), jnp.int32),
            pltpu.VMEM((B3,), jnp.int32),
            pltpu.VMEM((16,), jnp.float32),
            pltpu.VMEM((B3, 16), jnp.float32),
            pltpu.VMEM((B3, 16), jnp.float32),
            pltpu.VMEM((B3, SC3_W), jnp.float32),
            pltpu.VMEM((B3, SC3_W), jnp.float32),
            pltpu.SemaphoreType.DMA,
            pltpu.SemaphoreType.DMA,
            pltpu.SemaphoreType.DMA,
            pltpu.SemaphoreType.DMA,
            pltpu.VMEM_SHARED((SC3_ROWS, SC3_AW), jnp.float32),
        ],
    )(whel, dtab, srcp, dstp, mx16, z3)


def kernel(x, edge_index, edge_attr, Wf, bf, gf, betaf, We, be, ge, betae,
           Wfc, al, ar, bgat, g1, beta1, Wgate, bgate, Wo, bo, go, betao):
    f32 = jnp.float32
    xp = jnp.zeros((NP, 8), f32).at[:N, :7].set(x)
    Wf8 = jnp.zeros((8, H0), f32).at[:7].set(Wf)
    eap = jnp.zeros((EP, 8), f32).at[:E, :6].set(edge_attr)
    We8 = jnp.zeros((8, H0), f32).at[:6].set(We)
    srcp = jnp.concatenate([edge_index[0], jnp.zeros((EP - E,), jnp.int32)])
    dstp = jnp.concatenate([edge_index[1],
                            jnp.full((EP - E,), N, jnp.int32)])
    Al = jnp.zeros((H1, H), f32)
    Ar = jnp.zeros((H1, H), f32)
    for hh in range(H):
        Al = Al.at[hh * DH:(hh + 1) * DH, hh].set(al[hh])
        Ar = Ar.at[hh * DH:(hh + 1) * DH, hh].set(ar[hh])
    S4 = jnp.kron(jnp.eye(H, dtype=f32), jnp.ones((1, DH), f32))
    Wg8 = jnp.zeros((H1, 8), f32).at[:, 0:1].set(Wgate)
    bg8 = jnp.zeros((1, 8), f32).at[0, 0].set(bgate[0])
    r = lambda v: v.reshape(1, -1)
    z1 = jnp.zeros((SC1_STRIPE, SC1_W), f32)
    z3 = jnp.zeros((SC3_STRIPE, SC3_AW), f32)

    h = pl.pallas_call(
        _tc_face_body,
        grid=(NBLK,),
        in_specs=[
            pl.BlockSpec((RB, 8), lambda i: (i, 0)),
            pl.BlockSpec((8, H0), lambda i: (0, 0)),
            _row_spec(H0), _row_spec(H0), _row_spec(H0),
        ],
        out_specs=pl.BlockSpec((RB, H0), lambda i: (i, 0)),
        out_shape=jax.ShapeDtypeStruct((NP, H0), f32),
    )(xp, Wf8, r(bf), r(gf), r(betaf))

    eep = pl.pallas_call(
        _tc_edge_body,
        grid=(EBLK,),
        in_specs=[
            pl.BlockSpec((EB, 8), lambda i: (i, 0)),
            pl.BlockSpec((8, H0), lambda i: (0, 0)),
            _row_spec(H0), _row_spec(H0), _row_spec(H0),
        ],
        out_specs=pl.BlockSpec((EB, SC1_W), lambda i: (i, 0)),
        out_shape=jax.ShapeDtypeStruct((EP, SC1_W), f32),
    )(eap, We8, r(be), r(ge), r(betae))

    ssc = _sc1_call(eep, dstp, z1)
    sscp = jnp.zeros((NP, SC1_W), f32).at[:N].set(ssc)

    whel, dtab, mx16 = pl.pallas_call(
        _tc_mid_body,
        grid=(NBLK,),
        in_specs=[
            pl.BlockSpec((RB, H0), lambda i: (i, 0)),
            pl.BlockSpec((RB, SC1_W), lambda i: (i, 0)),
            pl.BlockSpec((H0, H1), lambda i: (0, 0)),
            pl.BlockSpec((H1, H), lambda i: (0, 0)),
            pl.BlockSpec((H1, H), lambda i: (0, 0)),
            _row_spec(H0), _row_spec(H0), _row_spec(H0),
        ],
        out_specs=[
            pl.BlockSpec((RB, SC3_W), lambda i: (i, 0)),
            pl.BlockSpec((RB, 16), lambda i: (i, 0)),
            pl.BlockSpec((1, 16), lambda i: (0, 0)),
        ],
        out_shape=[
            jax.ShapeDtypeStruct((NP, SC3_W), f32),
            jax.ShapeDtypeStruct((NP, 16), f32),
            jax.ShapeDtypeStruct((1, 16), f32),
        ],
    )(h, sscp, Wfc, Al, Ar, r(be), r(ge), r(betae))

    nd = _sc3_call(whel, dtab, srcp, dstp, mx16.reshape(16), z3)
    ndp = jnp.zeros((NP, SC3_AW), f32).at[:N].set(nd)

    nf, z8 = pl.pallas_call(
        _tc_post_body,
        grid=(NBLK,),
        in_specs=[
            pl.BlockSpec((RB, SC3_AW), lambda i: (i, 0)),
            pl.BlockSpec((RB, SC3_W), lambda i: (i, 0)),
            pl.BlockSpec((RB, 16), lambda i: (i, 0)),
            pl.BlockSpec((1, 16), lambda i: (0, 0)),
            pl.BlockSpec((H, H1), lambda i: (0, 0)),
            _row_spec(H1), _row_spec(H1), _row_spec(H1),
            pl.BlockSpec((H1, 8), lambda i: (0, 0)),
            _row_spec(8),
        ],
        out_specs=[
            pl.BlockSpec((RB, H1), lambda i: (i, 0)),
            pl.BlockSpec((RB, 8), lambda i: (i, 0)),
        ],
        out_shape=[
            jax.ShapeDtypeStruct((NP, H1), f32),
            jax.ShapeDtypeStruct((NP, 8), f32),
        ],
    )(ndp, whel, dtab, mx16, S4, r(bgat), r(g1), r(beta1), Wg8, bg8)

    zmax8 = pl.pallas_call(
        _tc_zmax_body,
        grid=(NBLK,),
        in_specs=[pl.BlockSpec((RB, 8), lambda i: (i, 0))],
        out_specs=pl.BlockSpec((1, 8), lambda i: (0, 0)),
        out_shape=jax.ShapeDtypeStruct((1, 8), f32),
    )(z8)

    pooled, feat = pl.pallas_call(
        _tc_pool_body,
        grid=(NBLK,),
        in_specs=[
            pl.BlockSpec((RB, H1), lambda i: (i, 0)),
            pl.BlockSpec((RB, 8), lambda i: (i, 0)),
            pl.BlockSpec((1, 8), lambda i: (0, 0)),
            pl.BlockSpec((H1, H1), lambda i: (0, 0)),
            _row_spec(H1), _row_spec(H1), _row_spec(H1),
        ],
        out_specs=[
            pl.BlockSpec((1, H1), lambda i: (0, 0)),
            pl.BlockSpec((1, H1), lambda i: (0, 0)),
        ],
        out_shape=[
            jax.ShapeDtypeStruct((1, H1), f32),
            jax.ShapeDtypeStruct((1, H1), f32),
        ],
        scratch_shapes=[
            pltpu.VMEM((1, H1), f32),
            pltpu.VMEM((1, 8), f32),
        ],
    )(nf, z8, zmax8, Wo, r(bo), r(go), r(betao))

    return (feat, nf[:N], pooled)


# speedup vs baseline: 1.5931x; 1.0204x over previous
import functools

import jax
import jax.numpy as jnp
from jax import lax
from jax.experimental import pallas as pl
from jax.experimental.pallas import tpu as pltpu
from jax.experimental.pallas import tpu_sc as plsc

N = 50000
E = 800000
H = 4
DH = 32
H0 = 64
H1 = 128

NP = 50176
EP = 819200
NBLK = 98
RB = 512
EB = 1024
EBLK = EP // EB

SC1_HALF = 25000
SC1_ROWS = 25088
SC1_STRIPE = 1568
SC1_W = 72
SC3_CHUNK = 12512
SC3_ROWS = 12513
SC3_STRIPE = 784
SC3_W = 136
SC3_AW = 136
EDGES_PER_TILE = EP // 16
B = 64
NBLOCKS_TILE = EDGES_PER_TILE // B
B3 = 80
NBLOCKS3_TILE = EDGES_PER_TILE // B3


def _ln(h, g, b, eps=1e-5):
    mu = jnp.mean(h, axis=-1, keepdims=True)
    va = jnp.mean((h - mu) * (h - mu), axis=-1, keepdims=True)
    return (h - mu) / jnp.sqrt(va + eps) * g + b


def _lrelu(v):
    return jnp.maximum(v, 0.2 * v)


def _tc_face_body(xb, Wf8, bf, gf, betaf, hb):
    y = jax.nn.relu(jnp.dot(xb[...], Wf8[...],
                            preferred_element_type=jnp.float32) + bf[...])
    hb[...] = _ln(y, gf[...], betaf[...])


def _tc_edge_body(eab, We8, be, ge, betae, eepb):
    y = jax.nn.relu(jnp.dot(eab[...], We8[...],
                            preferred_element_type=jnp.float32) + be[...])
    ee = _ln(y, ge[...], betae[...])
    ones = jnp.ones((EB, 1), jnp.float32)
    zeros = jnp.zeros((EB, SC1_W - H0 - 1), jnp.float32)
    eepb[...] = jnp.concatenate([ee, ones, zeros], axis=1)


def _tc_mid_body(hb, sscb, Wfc, Al, Ar, be, ge, betae, whelb, dtabb, mx16b):
    i = pl.program_id(0)
    c0 = _ln(jax.nn.relu(be[...]), ge[...], betae[...])
    ssum = sscb[:, :H0]
    cnt = sscb[:, H0:H0 + 1]
    h2 = hb[...] + (ssum + c0) / (cnt + 1.0)
    Wh = jnp.dot(h2, Wfc[...], preferred_element_type=jnp.float32)
    el = jnp.dot(Wh, Al[...], preferred_element_type=jnp.float32)
    er = jnp.dot(Wh, Ar[...], preferred_element_type=jnp.float32)
    z8 = jnp.zeros((RB, 8), jnp.float32)
    z4 = jnp.zeros((RB, 4), jnp.float32)
    whelb[...] = jnp.concatenate([Wh, el, z4], axis=1)
    dtabb[...] = jnp.concatenate([z8, er, z4], axis=1)
    elz = jnp.concatenate([z8, el, z4], axis=1)
    bm = jnp.max(elz, axis=0, keepdims=True)

    @pl.when(i == 0)
    def _():
        mx16b[...] = jnp.full((1, 16), -1e30, jnp.float32)

    mx16b[...] = jnp.maximum(mx16b[...], bm)


def _tc_post_body(ndb, whelb, dtabb, mx16b, S4, bgat, g1, beta1, Wg8, bg8,
                  nfb, z8b):
    Wh = whelb[:, :H1]
    elv = whelb[:, H1:H1 + H]
    erv = dtabb[:, 8:12]
    mx = mx16b[:, 8:12]
    sE = _lrelu(elv + erv)
    M = _lrelu(mx + erv)
    sw = jnp.exp(sE - M)
    num = ndb[:, :H1] + jnp.dot(sw, S4[...],
                                preferred_element_type=jnp.float32) * Wh
    den = ndb[:, H1:H1 + H] + sw
    out = num / (jnp.dot(den, S4[...],
                         preferred_element_type=jnp.float32) + 1e-9)
    out = jax.nn.relu(out + bgat[...])
    nf = _ln(out, g1[...], beta1[...])
    nfb[...] = nf
    z8b[...] = jnp.dot(nf, Wg8[...], preferred_element_type=jnp.float32) + bg8[...]


def _tc_zmax_body(z8b, zmaxb):
    i = pl.program_id(0)
    rows = lax.broadcasted_iota(jnp.int32, (RB, 1), 0) + i * RB
    zm = jnp.where(rows < N, z8b[:, 0:1], -1e30)
    bm = jnp.max(zm)

    @pl.when(i == 0)
    def _():
        zmaxb[...] = jnp.full((1, 8), -1e30, jnp.float32)

    zmaxb[...] = jnp.maximum(zmaxb[...], bm)


def _tc_pool_body(nfb, z8b, zmaxb, Wo, bo, go, betao, pooledb, featb,
                  Sv, S1):
    i = pl.program_id(0)

    @pl.when(i == 0)
    def _():
        Sv[...] = jnp.zeros((1, H1), jnp.float32)
        S1[...] = jnp.zeros((1, 8), jnp.float32)

    rows = lax.broadcasted_iota(jnp.int32, (RB, 1), 0) + i * RB
    ge_ = jnp.where(rows < N, jnp.exp(z8b[:, 0:1] - zmaxb[:, 0:1]), 0.0)
    Sv[...] += jnp.sum(ge_ * nfb[...], axis=0, keepdims=True)
    S1[...] += jnp.full((1, 8), jnp.sum(ge_), jnp.float32)

    @pl.when(i == NBLK - 1)
    def _():
        pooled = Sv[...] / S1[:, 0:1]
        feat = jax.nn.relu(jnp.dot(pooled, Wo[...],
                                   preferred_element_type=jnp.float32) + bo[...])
        featb[...] = _ln(feat, go[...], betao[...])
        pooledb[...] = pooled


def _row_spec(w):
    return pl.BlockSpec((1, w), lambda i: (0, 0))


_SC_MESH = plsc.VectorSubcoreMesh(core_axis_name="c", subcore_axis_name="s")
_SC_PARAMS = pltpu.CompilerParams(use_tc_tiling_on_sc=False)


def _sc1_body(eep_hbm, dst_hbm, z1_hbm, out_hbm,
              dstv0, dstv1, idxv0, idxv1, valv0, valv1, semV0, semV1, accum):
    c = lax.axis_index("c")
    s = lax.axis_index("s")
    base = c * SC1_HALF
    pltpu.sync_copy(z1_hbm, accum.at[pl.ds(s * SC1_STRIPE, SC1_STRIPE), :])
    plsc.subcore_barrier()

    tile_off = s * EDGES_PER_TILE
    bufs = ((dstv0, idxv0, valv0, semV0), (dstv1, idxv1, valv1, semV1))

    for b in range(2):
        dstv, idxv, valv, semV = bufs[b]
        off = tile_off + b * B
        pltpu.sync_copy(dst_hbm.at[pl.ds(off, B)], dstv)
        pltpu.async_copy(eep_hbm.at[pl.ds(off, B), :], valv, semV)

    @pl.loop(0, NBLOCKS_TILE, step=2)
    def _blk(g):
        for b in range(2):
            dstv, idxv, valv, semV = bufs[b]
            blk = g + b
            off = tile_off + blk * B
            pltpu.make_async_copy(
                eep_hbm.at[pl.ds(off, B), :], valv, semV).wait()
            for j in range(B // 16):
                d = dstv[pl.ds(j * 16, 16)]
                ok = (d >= base) & (d < base + SC1_HALF)
                idxv[pl.ds(j * 16, 16)] = jnp.where(ok, d - base, SC1_HALF)
            pltpu.sync_copy(valv, accum.at[idxv], add=True)
            nxt = blk + 2

            @pl.when(nxt < NBLOCKS_TILE)
            def _():
                off2 = tile_off + nxt * B
                pltpu.sync_copy(dst_hbm.at[pl.ds(off2, B)], dstv)
                pltpu.async_copy(eep_hbm.at[pl.ds(off2, B), :], valv, semV)

    plsc.subcore_barrier()
    @pl.when(s < 15)
    def _():
        pltpu.sync_copy(
            accum.at[pl.ds(s * SC1_STRIPE, SC1_STRIPE), :],
            out_hbm.at[pl.ds(base + s * SC1_STRIPE, SC1_STRIPE), :])

    @pl.when(s == 15)
    def _():
        pltpu.sync_copy(
            accum.at[pl.ds(15 * SC1_STRIPE, SC1_HALF - 15 * SC1_STRIPE), :],
            out_hbm.at[pl.ds(base + 15 * SC1_STRIPE,
                             SC1_HALF - 15 * SC1_STRIPE), :])


def _sc1_call(eep, dstp, z1):
    return pl.kernel(
        _sc1_body,
        out_type=jax.ShapeDtypeStruct((N, SC1_W), jnp.float32),
        mesh=_SC_MESH,
        compiler_params=_SC_PARAMS,
        scratch_types=[
            pltpu.VMEM((B,), jnp.int32),
            pltpu.VMEM((B,), jnp.int32),
            pltpu.VMEM((B,), jnp.int32),
            pltpu.VMEM((B,), jnp.int32),
            pltpu.VMEM((B, SC1_W), jnp.float32),
            pltpu.VMEM((B, SC1_W), jnp.float32),
            pltpu.SemaphoreType.DMA,
            pltpu.SemaphoreType.DMA,
            pltpu.VMEM_SHARED((SC1_ROWS, SC1_W), jnp.float32),
        ],
    )(eep, dstp, z1)


def _sc3_body(whel_hbm, dtab_hbm, src_hbm, dst_hbm, mx_hbm, z3_hbm, out_hbm,
              srcv0, srcv1, dstv0, dstv1, mxv,
              dbuf0, dbuf1, wbuf0, wbuf1,
              semA0, semA1, semB0, semB1, accum):
    c = lax.axis_index("c")
    s = lax.axis_index("s")
    tile_off = s * EDGES_PER_TILE
    pltpu.sync_copy(mx_hbm, mxv)
    mxvec = mxv[...]
    bufs = ((srcv0, dstv0, dbuf0, wbuf0, semA0, semB0),
            (srcv1, dstv1, dbuf1, wbuf1, semA1, semB1))

    for k in range(2):
        chunk = c * 2 + k
        base = chunk * SC3_CHUNK
        hi = jnp.minimum(base + SC3_CHUNK, N)

        @pl.when(s < 15)
        def _():
            pltpu.sync_copy(
                z3_hbm, accum.at[pl.ds(s * SC3_STRIPE, SC3_STRIPE), :])

        @pl.when(s == 15)
        def _():
            pltpu.sync_copy(
                z3_hbm.at[pl.ds(0, SC3_ROWS - 15 * SC3_STRIPE), :],
                accum.at[pl.ds(15 * SC3_STRIPE,
                               SC3_ROWS - 15 * SC3_STRIPE), :])

        plsc.subcore_barrier()

        for b in range(2):
            srcv, dstv, dbuf, wbuf, semA, semB = bufs[b]
            off = tile_off + b * B3
            pltpu.sync_copy(src_hbm.at[pl.ds(off, B3)], srcv)
            pltpu.sync_copy(dst_hbm.at[pl.ds(off, B3)], dstv)
            pltpu.async_copy(whel_hbm.at[srcv], wbuf, semA)
            pltpu.async_copy(dtab_hbm.at[dstv], dbuf, semB)

        @pl.loop(0, NBLOCKS3_TILE, step=2)
        def _blk(g):
            for b in range(2):
                srcv, dstv, dbuf, wbuf, semA, semB = bufs[b]
                blk = g + b
                pltpu.make_async_copy(whel_hbm.at[srcv], wbuf, semA).wait()
                pltpu.make_async_copy(dtab_hbm.at[dstv], dbuf, semB).wait()

                lane = lax.iota(jnp.int32, 16)
                lo8 = lane < 8

                @pl.loop(0, B3, unroll=8)
                def _edge(j):
                    elv = wbuf[j, pl.ds(120, 16)]
                    erv = dbuf[j, pl.ds(0, 16)]
                    e_ = _lrelu(elv + erv)
                    M_ = _lrelu(mxvec + erv)
                    wv = jnp.exp(e_ - M_)
                    for hh in range(H):
                        bv = jnp.broadcast_to(wv[8 + hh], (16,))
                        for t in range(2):
                            col = hh * 32 + t * 16
                            wbuf[j, pl.ds(col, 16)] = bv * wbuf[j, pl.ds(col, 16)]
                    cur = wbuf[j, pl.ds(120, 16)]
                    wbuf[j, pl.ds(120, 16)] = jnp.where(lo8, cur, wv)

                for jj in range(B3 // 16):
                    d = dstv[pl.ds(jj * 16, 16)]
                    ok = (d >= base) & (d < hi)
                    dstv[pl.ds(jj * 16, 16)] = jnp.where(ok, d - base,
                                                         SC3_CHUNK)
                pltpu.sync_copy(wbuf, accum.at[dstv], add=True)

                nxt = blk + 2

                @pl.when(nxt < NBLOCKS3_TILE)
                def _():
                    off2 = tile_off + nxt * B3
                    pltpu.sync_copy(src_hbm.at[pl.ds(off2, B3)], srcv)
                    pltpu.sync_copy(dst_hbm.at[pl.ds(off2, B3)], dstv)
                    pltpu.async_copy(whel_hbm.at[srcv], wbuf, semA)
                    pltpu.async_copy(dtab_hbm.at[dstv], dbuf, semB)

        plsc.subcore_barrier()

        @pl.when(s < 15)
        def _():
            pltpu.sync_copy(
                accum.at[pl.ds(s * SC3_STRIPE, SC3_STRIPE), :],
                out_hbm.at[pl.ds(base + s * SC3_STRIPE, SC3_STRIPE), :])

        full15 = 15 * SC3_STRIPE
        if k == 0:
            @pl.when(s == 15)
            def _():
                pltpu.sync_copy(
                    accum.at[pl.ds(full15, SC3_CHUNK - full15), :],
                    out_hbm.at[pl.ds(base + full15, SC3_CHUNK - full15), :])
        else:
            @pl.when((s == 15) & (c == 0))
            def _():
                pltpu.sync_copy(
                    accum.at[pl.ds(full15, SC3_CHUNK - full15), :],
                    out_hbm.at[pl.ds(base + full15, SC3_CHUNK - full15), :])

            @pl.when((s == 15) & (c == 1))
            def _():
                pltpu.sync_copy(
                    accum.at[pl.ds(full15, N - 3 * SC3_CHUNK - full15), :],
                    out_hbm.at[pl.ds(base + full15,
                                     N - 3 * SC3_CHUNK - full15), :])

        plsc.subcore_barrier()


def _sc3_call(whel, dtab, srcp, dstp, mx16, z3):
    return pl.kernel(
        _sc3_body,
        out_type=jax.ShapeDtypeStruct((N, SC3_AW), jnp.float32),
        mesh=_SC_MESH,
        compiler_params=_SC_PARAMS,
        scratch_types=[
            pltpu.VMEM((B3,), jnp.int32),
            pltpu.VMEM((B3,), jnp.int32),
            pltpu.VMEM((B3,), jnp.int32),
            pltpu.VMEM((B3,), jnp.int32),
            pltpu.VMEM((16,), jnp.float32),
            pltpu.VMEM((B3, 16), jnp.float32),
            pltpu.VMEM((B3, 16), jnp.float32),
            pltpu.VMEM((B3, SC3_W), jnp.float32),
            pltpu.VMEM((B3, SC3_W), jnp.float32),
            pltpu.SemaphoreType.DMA,
            pltpu.SemaphoreType.DMA,
            pltpu.SemaphoreType.DMA,
            pltpu.SemaphoreType.DMA,
            pltpu.VMEM_SHARED((SC3_ROWS, SC3_AW), jnp.float32),
        ],
    )(whel, dtab, srcp, dstp, mx16, z3)


def kernel(x, edge_index, edge_attr, Wf, bf, gf, betaf, We, be, ge, betae,
           Wfc, al, ar, bgat, g1, beta1, Wgate, bgate, Wo, bo, go, betao):
    f32 = jnp.float32
    xp = jnp.zeros((NP, 8), f32).at[:N, :7].set(x)
    Wf8 = jnp.zeros((8, H0), f32).at[:7].set(Wf)
    eap = jnp.zeros((EP, 8), f32).at[:E, :6].set(edge_attr)
    We8 = jnp.zeros((8, H0), f32).at[:6].set(We)
    srcp = jnp.concatenate([edge_index[0], jnp.zeros((EP - E,), jnp.int32)])
    dstp = jnp.concatenate([edge_index[1],
                            jnp.full((EP - E,), N, jnp.int32)])
    Al = jnp.zeros((H1, H), f32)
    Ar = jnp.zeros((H1, H), f32)
    for hh in range(H):
        Al = Al.at[hh * DH:(hh + 1) * DH, hh].set(al[hh])
        Ar = Ar.at[hh * DH:(hh + 1) * DH, hh].set(ar[hh])
    S4 = jnp.kron(jnp.eye(H, dtype=f32), jnp.ones((1, DH), f32))
    Wg8 = jnp.zeros((H1, 8), f32).at[:, 0:1].set(Wgate)
    bg8 = jnp.zeros((1, 8), f32).at[0, 0].set(bgate[0])
    r = lambda v: v.reshape(1, -1)
    z1 = jnp.zeros((SC1_STRIPE, SC1_W), f32)
    z3 = jnp.zeros((SC3_STRIPE, SC3_AW), f32)

    h = pl.pallas_call(
        _tc_face_body,
        grid=(NBLK,),
        in_specs=[
            pl.BlockSpec((RB, 8), lambda i: (i, 0)),
            pl.BlockSpec((8, H0), lambda i: (0, 0)),
            _row_spec(H0), _row_spec(H0), _row_spec(H0),
        ],
        out_specs=pl.BlockSpec((RB, H0), lambda i: (i, 0)),
        out_shape=jax.ShapeDtypeStruct((NP, H0), f32),
    )(xp, Wf8, r(bf), r(gf), r(betaf))

    eep = pl.pallas_call(
        _tc_edge_body,
        grid=(EBLK,),
        in_specs=[
            pl.BlockSpec((EB, 8), lambda i: (i, 0)),
            pl.BlockSpec((8, H0), lambda i: (0, 0)),
            _row_spec(H0), _row_spec(H0), _row_spec(H0),
        ],
        out_specs=pl.BlockSpec((EB, SC1_W), lambda i: (i, 0)),
        out_shape=jax.ShapeDtypeStruct((EP, SC1_W), f32),
    )(eap, We8, r(be), r(ge), r(betae))

    ssc = _sc1_call(eep, dstp, z1)
    sscp = jnp.zeros((NP, SC1_W), f32).at[:N].set(ssc)

    whel, dtab, mx16 = pl.pallas_call(
        _tc_mid_body,
        grid=(NBLK,),
        in_specs=[
            pl.BlockSpec((RB, H0), lambda i: (i, 0)),
            pl.BlockSpec((RB, SC1_W), lambda i: (i, 0)),
            pl.BlockSpec((H0, H1), lambda i: (0, 0)),
            pl.BlockSpec((H1, H), lambda i: (0, 0)),
            pl.BlockSpec((H1, H), lambda i: (0, 0)),
            _row_spec(H0), _row_spec(H0), _row_spec(H0),
        ],
        out_specs=[
            pl.BlockSpec((RB, SC3_W), lambda i: (i, 0)),
            pl.BlockSpec((RB, 16), lambda i: (i, 0)),
            pl.BlockSpec((1, 16), lambda i: (0, 0)),
        ],
        out_shape=[
            jax.ShapeDtypeStruct((NP, SC3_W), f32),
            jax.ShapeDtypeStruct((NP, 16), f32),
            jax.ShapeDtypeStruct((1, 16), f32),
        ],
    )(h, sscp, Wfc, Al, Ar, r(be), r(ge), r(betae))

    nd = _sc3_call(whel, dtab, srcp, dstp, mx16.reshape(16), z3)
    ndp = jnp.zeros((NP, SC3_AW), f32).at[:N].set(nd)

    nf, z8 = pl.pallas_call(
        _tc_post_body,
        grid=(NBLK,),
        in_specs=[
            pl.BlockSpec((RB, SC3_AW), lambda i: (i, 0)),
            pl.BlockSpec((RB, SC3_W), lambda i: (i, 0)),
            pl.BlockSpec((RB, 16), lambda i: (i, 0)),
            pl.BlockSpec((1, 16), lambda i: (0, 0)),
            pl.BlockSpec((H, H1), lambda i: (0, 0)),
            _row_spec(H1), _row_spec(H1), _row_spec(H1),
            pl.BlockSpec((H1, 8), lambda i: (0, 0)),
            _row_spec(8),
        ],
        out_specs=[
            pl.BlockSpec((RB, H1), lambda i: (i, 0)),
            pl.BlockSpec((RB, 8), lambda i: (i, 0)),
        ],
        out_shape=[
            jax.ShapeDtypeStruct((NP, H1), f32),
            jax.ShapeDtypeStruct((NP, 8), f32),
        ],
    )(ndp, whel, dtab, mx16, S4, r(bgat), r(g1), r(beta1), Wg8, bg8)

    zmax8 = pl.pallas_call(
        _tc_zmax_body,
        grid=(NBLK,),
        in_specs=[pl.BlockSpec((RB, 8), lambda i: (i, 0))],
        out_specs=pl.BlockSpec((1, 8), lambda i: (0, 0)),
        out_shape=jax.ShapeDtypeStruct((1, 8), f32),
    )(z8)

    pooled, feat = pl.pallas_call(
        _tc_pool_body,
        grid=(NBLK,),
        in_specs=[
            pl.BlockSpec((RB, H1), lambda i: (i, 0)),
            pl.BlockSpec((RB, 8), lambda i: (i, 0)),
            pl.BlockSpec((1, 8), lambda i: (0, 0)),
            pl.BlockSpec((H1, H1), lambda i: (0, 0)),
            _row_spec(H1), _row_spec(H1), _row_spec(H1),
        ],
        out_specs=[
            pl.BlockSpec((1, H1), lambda i: (0, 0)),
            pl.BlockSpec((1, H1), lambda i: (0, 0)),
        ],
        out_shape=[
            jax.ShapeDtypeStruct((1, H1), f32),
            jax.ShapeDtypeStruct((1, H1), f32),
        ],
        scratch_shapes=[
            pltpu.VMEM((1, H1), f32),
            pltpu.VMEM((1, 8), f32),
        ],
    )(nf, z8, zmax8, Wo, r(bo), r(go), r(betao))

    return (feat, nf[:N], pooled)
